# Initial kernel scaffold; baseline (speedup 1.0000x reference)
import jax
import jax.numpy as jnp
from jax.experimental import pallas as pl


def kernel(pos, edge_index, batch, W1_l, W1_r, b1, W2_l, W2_r, b2, W_gat, att_src, att_dst, b_gat, Wd1, bd1, g1, be1, Wd2, bd2, g2, be2, Wd3, bd3, g3, be3):
    raise NotImplementedError("write your pallas kernel here")


# speedup vs baseline: 13.8754x; 13.8754x over previous
import jax
import jax.numpy as jnp
from jax import lax
from jax.experimental import pallas as pl
from jax.experimental.pallas import tpu as pltpu
from jax.experimental.pallas import tpu_sc as plsc

N = 10000
E = 320000
D = 128
HD = D // 2
B = 16
HEADS = 2
EPS = 1e-5
OUT = 40

NC = 2
NS = 16
NW = NC * NS

K = 80
EPW = E // NW
EPC = E // NS
GAT_CH = EPC // K
SLAB = 640
SLAB_LAST = N - SLAB * (NS - 1)
ZR = 40

_mesh = plsc.VectorSubcoreMesh(
    core_axis_name="c", subcore_axis_name="s", num_cores=NC, num_subcores=NS)
_sc_params = pltpu.CompilerParams(needs_layout_passes=False,
                                  use_tc_tiling_on_sc=False)


def _fill_zero(zbuf):
  for i in range(ZR):
    for q in range(HD // 16):
      zbuf[i, pl.ds(q * 16, 16)] = jnp.zeros((16,), jnp.float32)


def _zero_rows(zbuf, accum, sid):
  @pl.when(sid < NS - 1)
  def _():
    for k in range(SLAB // ZR):
      pltpu.sync_copy(zbuf, accum.at[pl.ds(sid * SLAB + k * ZR, ZR)])

  @pl.when(sid == NS - 1)
  def _():
    for k in range(SLAB_LAST // ZR):
      pltpu.sync_copy(zbuf, accum.at[pl.ds((NS - 1) * SLAB + k * ZR, ZR)])


def _writeout_rows(accum, out_hbm, sid, roff):
  @pl.when(sid < NS - 1)
  def _():
    pltpu.sync_copy(accum.at[pl.ds(sid * SLAB, SLAB)],
                    out_hbm.at[pl.ds(roff + sid * SLAB, SLAB)])

  @pl.when(sid == NS - 1)
  def _():
    pltpu.sync_copy(accum.at[pl.ds((NS - 1) * SLAB, SLAB_LAST)],
                    out_hbm.at[pl.ds(roff + (NS - 1) * SLAB, SLAB_LAST)])


def _zero_table(tab):
  def body(i, _):
    tab[pl.ds(i * 16, 16)] = jnp.zeros((16,), jnp.float32)
    return 0

  lax.fori_loop(0, N // 16, body, 0)


def _make_sage(with_deg):
  out_type = [jax.ShapeDtypeStruct((NC * 2 * N, HD), jnp.float32)]
  if with_deg:
    out_type.append(jax.ShapeDtypeStruct((NW * N,), jnp.float32))
  scratch = [
      pltpu.VMEM((K,), jnp.int32),
      pltpu.VMEM((1, K), jnp.int32),
      pltpu.VMEM((K, HD), jnp.float32),
      pltpu.VMEM((ZR, HD), jnp.float32),
      pltpu.VMEM_SHARED((N, HD), jnp.float32),
      pltpu.SemaphoreType.DMA,
  ]
  if with_deg:
    scratch.append(pltpu.VMEM((N,), jnp.float32))

  def body(x_hbm, src_hbm, dst_hbm, p_out, *rest):
    if with_deg:
      deg_out, srcb, dstb, rows, zbuf, accum, sem, degt = rest
    else:
      srcb, dstb, rows, zbuf, accum, sem = rest
    cid = lax.axis_index("c")
    sid = lax.axis_index("s")
    wid = cid * NS + sid

    _fill_zero(zbuf)
    if with_deg:
      _zero_table(degt)
    ones16 = jnp.ones((16,), jnp.float32)

    for f in range(2):
      _zero_rows(zbuf, accum, sid)
      plsc.subcore_barrier()

      def chunk(ch, _):
        base = wid * EPW + ch * K
        pltpu.sync_copy(src_hbm.at[pl.ds(base, K)], srcb)
        pltpu.sync_copy(dst_hbm.at[pl.ds(base, K)], dstb.at[0])
        if f > 0:
          for k in range(K // 16):
            srcb[pl.ds(k * 16, 16)] = srcb[pl.ds(k * 16, 16)] + f * N
        pltpu.async_copy(x_hbm.at[srcb], rows, sem).wait()
        pltpu.sync_copy(rows, accum.at[dstb.at[0]], add=True)
        if with_deg and f == 0:
          for k in range(K // 16):
            d16 = dstb[0, pl.ds(k * 16, 16)]
            plsc.addupdate_scatter(degt, [d16], ones16)
        return 0

      lax.fori_loop(0, EPW // K, chunk, 0)
      plsc.subcore_barrier()

      _writeout_rows(accum, p_out, sid, (cid * 2 + f) * N)
      if f == 0:
        plsc.subcore_barrier()
    if with_deg:
      pltpu.sync_copy(degt, deg_out.at[pl.ds(wid * N, N)])

  return pl.kernel(body, out_type=tuple(out_type), mesh=_mesh,
                   scratch_types=tuple(scratch), compiler_params=_sc_params)


_sage_deg = _make_sage(True)
_sage = _make_sage(False)


def _gat_kernel():
  out_type = (jax.ShapeDtypeStruct((4 * N, HD), jnp.float32),
              jax.ShapeDtypeStruct((NW * N,), jnp.float32))
  scratch = (
      pltpu.VMEM((4 * N,), jnp.float32),
      pltpu.VMEM((K,), jnp.int32),
      pltpu.VMEM((1, K), jnp.int32),
      pltpu.VMEM((K, HD), jnp.float32),
      pltpu.VMEM((K,), jnp.float32),
      pltpu.VMEM((ZR, HD), jnp.float32),
      pltpu.VMEM((N,), jnp.float32),
      pltpu.VMEM_SHARED((N, HD), jnp.float32),
      pltpu.SemaphoreType.DMA,
  )

  def body(h_hbm, src_hbm, dst_hbm, asd_hbm, o_out, den_out,
           asd_v, srcb, dstb, rows, eeb, zbuf, dent, accum, sem):
    cid = lax.axis_index("c")
    sid = lax.axis_index("s")
    wid = cid * NS + sid

    pltpu.sync_copy(asd_hbm, asd_v)
    _fill_zero(zbuf)
    _zero_table(dent)

    col_s = jnp.full((16,), cid, jnp.int32)
    col_d = jnp.full((16,), 2 + cid, jnp.int32)
    iota16 = lax.iota(jnp.int32, 16)

    def mx_body(i, mv):
      v = plsc.load_gather(asd_v, [(iota16 + i * 16) * 4 + col_s])
      return jnp.maximum(mv, v)

    mv = lax.fori_loop(0, N // 16, mx_body,
                       jnp.full((16,), -3.4e38, jnp.float32))
    msv = jnp.full((16,), jnp.max(mv, axis=0), jnp.float32)

    for f in range(2):
      _zero_rows(zbuf, accum, sid)
      plsc.subcore_barrier()
      roff = (cid * 2 + f) * N

      def chunk(ch, _):
        base = sid * EPC + ch * K
        pltpu.sync_copy(src_hbm.at[pl.ds(base, K)], srcb)
        pltpu.sync_copy(dst_hbm.at[pl.ds(base, K)], dstb.at[0])
        for k in range(K // 16):
          s16 = srcb[pl.ds(k * 16, 16)]
          d16 = dstb[0, pl.ds(k * 16, 16)]
          es = plsc.load_gather(asd_v, [s16 * 4 + col_s])
          ad = plsc.load_gather(asd_v, [d16 * 4 + col_d])
          t = es + ad
          e = jnp.where(t > 0, t, 0.2 * t)
          c0 = msv + ad
          cc = jnp.where(c0 > 0, c0, 0.2 * c0)
          ee = jnp.exp(e - cc)
          if f == 0:
            plsc.addupdate_scatter(dent, [d16], ee)
          eeb[pl.ds(k * 16, 16)] = ee
          srcb[pl.ds(k * 16, 16)] = s16 + roff
        pltpu.async_copy(h_hbm.at[srcb], rows, sem).wait()

        def scale(r, _):
          av = plsc.load_gather(eeb, [jnp.full((16,), r, jnp.int32)])
          for q in range(HD // 16):
            rows[r, pl.ds(q * 16, 16)] = rows[r, pl.ds(q * 16, 16)] * av
          return 0

        lax.fori_loop(0, K, scale, 0)
        pltpu.sync_copy(rows, accum.at[dstb.at[0]], add=True)
        return 0

      lax.fori_loop(0, GAT_CH, chunk, 0)
      plsc.subcore_barrier()

      _writeout_rows(accum, o_out, sid, roff)
      if f == 0:
        plsc.subcore_barrier()
    pltpu.sync_copy(dent, den_out.at[pl.ds(wid * N, N)])

  return pl.kernel(body, out_type=out_type, mesh=_mesh,
                   scratch_types=scratch, compiler_params=_sc_params)


_gat = _gat_kernel()


R = 1000
GRID = N // R
_f32 = jnp.float32


def _cat(a, b):
  return jnp.concatenate([a, b], axis=-1)


def _tc1_body(pos, p, degp, wl, wr, b, x1):
  d = jnp.sum(degp[...], axis=0)
  cnt = jnp.maximum(d, 1.0)
  aggr = _cat(p[0] + p[2], p[1] + p[3]) / cnt
  posv = _cat(pos[0], pos[1])
  y = (jnp.dot(aggr, wl[...], preferred_element_type=_f32)
       + jnp.dot(posv, wr[...], preferred_element_type=_f32) + b[...])
  y = jnp.maximum(y, 0.0)
  x1[0] = y[:, :HD]
  x1[1] = y[:, HD:]


def _tc2_body(x1, q, degp, wl, wr, b, wg, ats, atd, h, asd):
  d = jnp.sum(degp[...], axis=0)
  cnt = jnp.maximum(d, 1.0)
  aggr = _cat(q[0] + q[2], q[1] + q[3]) / cnt
  x1v = _cat(x1[0], x1[1])
  x2 = jnp.maximum(
      jnp.dot(aggr, wl[...], preferred_element_type=_f32)
      + jnp.dot(x1v, wr[...], preferred_element_type=_f32) + b[...], 0.0)
  wgv = wg[...]
  hcat = (jnp.dot(x1v, wgv[:D, :], preferred_element_type=_f32)
          + jnp.dot(x2, wgv[D:, :], preferred_element_type=_f32))
  h0 = hcat[:, :D]
  h1 = hcat[:, D:]
  for i in range(4):
    h[i] = hcat[:, i * HD:(i + 1) * HD]
  atsv = ats[...]
  atdv = atd[...]
  asd[:, 0:1] = jnp.sum(h0 * atsv[0:1, :], axis=-1, keepdims=True)
  asd[:, 1:2] = jnp.sum(h1 * atsv[1:2, :], axis=-1, keepdims=True)
  asd[:, 2:3] = jnp.sum(h0 * atdv[0:1, :], axis=-1, keepdims=True)
  asd[:, 3:4] = jnp.sum(h1 * atdv[1:2, :], axis=-1, keepdims=True)


_BN_SCALE = float(1.0 / (1.0 + EPS) ** 0.5)


def _tc3_body(o, denp, bg, batch, wd1, bd1, g1, be1, wd2, bd2, g2, be2,
              wd3, bd3, g3, be3, z, am, gs, gc):
  i = pl.program_id(0)
  den0 = jnp.sum(denp[0], axis=0)
  den1 = jnp.sum(denp[1], axis=0)
  o0 = _cat(o[0], o[1])
  o1 = _cat(o[2], o[3])
  out = 0.5 * (o0 / (den0 + 1e-16) + o1 / (den1 + 1e-16)) + bg[...]
  gid = lax.broadcasted_iota(jnp.int32, (R, B), 1)
  oh = (batch[...] == gid).astype(_f32)
  gs_inc = lax.dot_general(oh, out, (((0,), (0,)), ((), ())),
                           preferred_element_type=_f32)
  gc_inc = lax.dot_general(oh, jnp.ones((R, 1), _f32),
                           (((0,), (0,)), ((), ())),
                           preferred_element_type=_f32)

  @pl.when(i == 0)
  def _():
    gs[...] = jnp.zeros_like(gs)
    gc[...] = jnp.zeros_like(gc)

  gs[...] += gs_inc
  gc[...] += gc_inc

  @pl.when(i == GRID - 1)
  def _():
    zv = gs[...] / jnp.maximum(gc[...], 1.0)

    def bn(x, g, bb):
      return x * _BN_SCALE * g[...] + bb[...]

    y = jnp.maximum(
        bn(jnp.dot(zv, wd1[...], preferred_element_type=_f32) + bd1[...],
           g1, be1), 0.0)
    y = jnp.maximum(
        bn(jnp.dot(y, wd2[...], preferred_element_type=_f32) + bd2[...],
           g2, be2), 0.0)
    y = bn(jnp.dot(y, wd3[...], preferred_element_type=_f32) + bd3[...],
           g3, be3)
    z[...] = y
    mx = jnp.max(y, axis=-1, keepdims=True)
    ii = lax.broadcasted_iota(jnp.int32, (B, OUT), 1)
    am[...] = jnp.min(jnp.where(y >= mx, ii, OUT), axis=-1, keepdims=True)


def _full(shape):
  return pl.BlockSpec(shape, lambda i: tuple(0 for _ in shape))


_tc1 = pl.pallas_call(
    _tc1_body,
    grid=(GRID,),
    in_specs=[
        pl.BlockSpec((2, R, HD), lambda i: (0, i, 0)),
        pl.BlockSpec((NC * 2, R, HD), lambda i: (0, i, 0)),
        pl.BlockSpec((NW, R, 1), lambda i: (0, i, 0)),
        _full((D, D)), _full((D, D)), _full((1, D)),
    ],
    out_specs=pl.BlockSpec((2, R, HD), lambda i: (0, i, 0)),
    out_shape=jax.ShapeDtypeStruct((2, N, HD), _f32),
)

_tc2 = pl.pallas_call(
    _tc2_body,
    grid=(GRID,),
    in_specs=[
        pl.BlockSpec((2, R, HD), lambda i: (0, i, 0)),
        pl.BlockSpec((NC * 2, R, HD), lambda i: (0, i, 0)),
        pl.BlockSpec((NW, R, 1), lambda i: (0, i, 0)),
        _full((D, D)), _full((D, D)), _full((1, D)),
        _full((2 * D, 2 * D)), _full((HEADS, D)), _full((HEADS, D)),
    ],
    out_specs=[
        pl.BlockSpec((4, R, HD), lambda i: (0, i, 0)),
        pl.BlockSpec((R, 4), lambda i: (i, 0)),
    ],
    out_shape=[
        jax.ShapeDtypeStruct((4, N, HD), _f32),
        jax.ShapeDtypeStruct((N, 4), _f32),
    ],
)

_tc3 = pl.pallas_call(
    _tc3_body,
    grid=(GRID,),
    in_specs=[
        pl.BlockSpec((4, R, HD), lambda i: (0, i, 0)),
        pl.BlockSpec((HEADS, NS, R, 1), lambda i: (0, 0, i, 0)),
        _full((1, D)),
        pl.BlockSpec((R, 1), lambda i: (i, 0)),
        _full((D, 2 * D)), _full((1, 2 * D)), _full((1, 2 * D)),
        _full((1, 2 * D)),
        _full((2 * D, D)), _full((1, D)), _full((1, D)), _full((1, D)),
        _full((D, OUT)), _full((1, OUT)), _full((1, OUT)), _full((1, OUT)),
    ],
    out_specs=[
        pl.BlockSpec((B, OUT), lambda i: (0, 0)),
        pl.BlockSpec((B, 1), lambda i: (0, 0)),
    ],
    out_shape=[
        jax.ShapeDtypeStruct((B, OUT), _f32),
        jax.ShapeDtypeStruct((B, 1), jnp.int32),
    ],
    scratch_shapes=[
        pltpu.VMEM((B, D), _f32),
        pltpu.VMEM((B, 1), _f32),
    ],
    compiler_params=pltpu.CompilerParams(
        dimension_semantics=("arbitrary",)),
)


def kernel(pos, edge_index, batch, W1_l, W1_r, b1, W2_l, W2_r, b2, W_gat,
           att_src, att_dst, b_gat, Wd1, bd1, g1, be1, Wd2, bd2, g2, be2,
           Wd3, bd3, g3, be3):
  src = edge_index[0]
  dst = edge_index[1]

  pos2 = pos.reshape(N, 2, HD).transpose(1, 0, 2)

  p, degp = _sage_deg(pos2.reshape(2 * N, HD), src, dst)
  p = p.reshape(NC * 2, N, HD)
  degp3 = degp.reshape(NW, N, 1)
  x1 = _tc1(pos2, p, degp3, W1_l, W1_r, b1.reshape(1, D))

  q = _sage(x1.reshape(2 * N, HD), src, dst)[0].reshape(NC * 2, N, HD)
  h, asd = _tc2(x1, q, degp3, W2_l, W2_r, b2.reshape(1, D),
                W_gat, att_src, att_dst)

  o, denp = _gat(h.reshape(4 * N, HD), src, dst, asd.reshape(4 * N))
  o = o.reshape(4, N, HD)
  denp4 = denp.reshape(HEADS, NS, N, 1)

  z, am = _tc3(o, denp4, b_gat.reshape(1, D), batch.reshape(N, 1),
               Wd1, bd1.reshape(1, -1), g1.reshape(1, -1), be1.reshape(1, -1),
               Wd2, bd2.reshape(1, -1), g2.reshape(1, -1), be2.reshape(1, -1),
               Wd3, bd3.reshape(1, -1), g3.reshape(1, -1), be3.reshape(1, -1))
  return (z, am.reshape(B))


# speedup vs baseline: 25.7118x; 1.8531x over previous
import jax
import jax.numpy as jnp
from jax import lax
from jax.experimental import pallas as pl
from jax.experimental.pallas import tpu as pltpu
from jax.experimental.pallas import tpu_sc as plsc

N = 10000
E = 320000
D = 128
HD = D // 2
B = 16
HEADS = 2
EPS = 1e-5
OUT = 40

NC = 2
NS = 16
NW = NC * NS

K = 80
EPW = E // NW
EPC = E // NS
GAT_CH = EPC // K
SLAB = 640
SLAB_LAST = N - SLAB * (NS - 1)
ZR = 40

_mesh = plsc.VectorSubcoreMesh(
    core_axis_name="c", subcore_axis_name="s", num_cores=NC, num_subcores=NS)
_sc_params = pltpu.CompilerParams(needs_layout_passes=False,
                                  use_tc_tiling_on_sc=False)


def _fill_zero(zbuf):
  w = zbuf.shape[-1]
  for i in range(ZR):
    for q in range(w // 16):
      zbuf[i, pl.ds(q * 16, 16)] = jnp.zeros((16,), jnp.float32)


def _zero_rows(zbuf, accum, sid):
  @pl.when(sid < NS - 1)
  def _():
    for k in range(SLAB // ZR):
      pltpu.sync_copy(zbuf, accum.at[pl.ds(sid * SLAB + k * ZR, ZR)])

  @pl.when(sid == NS - 1)
  def _():
    for k in range(SLAB_LAST // ZR):
      pltpu.sync_copy(zbuf, accum.at[pl.ds((NS - 1) * SLAB + k * ZR, ZR)])


def _writeout_rows(accum, out_hbm, sid, roff):
  @pl.when(sid < NS - 1)
  def _():
    pltpu.sync_copy(accum.at[pl.ds(sid * SLAB, SLAB)],
                    out_hbm.at[pl.ds(roff + sid * SLAB, SLAB)])

  @pl.when(sid == NS - 1)
  def _():
    pltpu.sync_copy(accum.at[pl.ds((NS - 1) * SLAB, SLAB_LAST)],
                    out_hbm.at[pl.ds(roff + (NS - 1) * SLAB, SLAB_LAST)])


def _zero_table(tab):
  def body(i, _):
    tab[pl.ds(i * 16, 16)] = jnp.zeros((16,), jnp.float32)
    return 0

  lax.fori_loop(0, N // 16, body, 0)


SAGE_CH = EPW // K


def _make_sage(with_deg):
  out_type = [jax.ShapeDtypeStruct((NC * N, D), jnp.float32)]
  if with_deg:
    out_type.append(jax.ShapeDtypeStruct((NW * N,), jnp.float32))
  scratch = [
      pltpu.VMEM((SAGE_CH, K), jnp.int32),
      pltpu.VMEM((K,), jnp.int32),
      pltpu.VMEM((K,), jnp.int32),
      pltpu.VMEM((K, D), jnp.float32),
      pltpu.VMEM((K, D), jnp.float32),
      pltpu.VMEM((ZR, D), jnp.float32),
      pltpu.VMEM_SHARED((N, D), jnp.float32),
      pltpu.SemaphoreType.DMA,
      pltpu.SemaphoreType.DMA,
  ]
  if with_deg:
    scratch.append(pltpu.VMEM((N,), jnp.float32))

  def body(x_hbm, src_hbm, dst2_hbm, p_out, *rest):
    if with_deg:
      (deg_out, dst2d, srcb0, srcb1, rows0, rows1, zbuf, accum,
       sem0, sem1, degt) = rest
    else:
      (dst2d, srcb0, srcb1, rows0, rows1, zbuf, accum, sem0, sem1) = rest
    cid = lax.axis_index("c")
    sid = lax.axis_index("s")
    wid = cid * NS + sid

    pltpu.sync_copy(dst2_hbm.at[pl.ds(wid * SAGE_CH, SAGE_CH)], dst2d)
    _fill_zero(zbuf)
    if with_deg:
      _zero_table(degt)
    ones16 = jnp.ones((16,), jnp.float32)

    _zero_rows(zbuf, accum, sid)
    plsc.subcore_barrier()

    def stage(ch, srcb, rows, sem):
      pltpu.sync_copy(src_hbm.at[pl.ds(wid * EPW + ch * K, K)], srcb)
      pltpu.async_copy(x_hbm.at[srcb], rows, sem)

    def drain(ch, rows, sem):
      pltpu.make_async_copy(x_hbm.at[pl.ds(0, K)], rows, sem).wait()
      pltpu.sync_copy(rows, accum.at[dst2d.at[ch]], add=True)
      if with_deg:
        for k in range(K // 16):
          d16 = dst2d[ch, pl.ds(k * 16, 16)]
          plsc.addupdate_scatter(degt, [d16], ones16)

    stage(0, srcb0, rows0, sem0)

    def pair(j, _):
      stage(2 * j + 1, srcb1, rows1, sem1)
      drain(2 * j, rows0, sem0)
      stage(2 * j + 2, srcb0, rows0, sem0)
      drain(2 * j + 1, rows1, sem1)
      return 0

    lax.fori_loop(0, SAGE_CH // 2, pair, 0)
    drain(SAGE_CH - 1, rows0, sem0)
    plsc.subcore_barrier()

    _writeout_rows(accum, p_out, sid, cid * N)
    if with_deg:
      pltpu.sync_copy(degt, deg_out.at[pl.ds(wid * N, N)])

  return pl.kernel(body, out_type=tuple(out_type), mesh=_mesh,
                   scratch_types=tuple(scratch), compiler_params=_sc_params)


_sage_deg = _make_sage(True)
_sage = _make_sage(False)


def _gat_kernel():
  out_type = (jax.ShapeDtypeStruct((4 * N, HD), jnp.float32),
              jax.ShapeDtypeStruct((NW * N,), jnp.float32))
  scratch = (
      pltpu.VMEM((4 * N,), jnp.float32),
      pltpu.VMEM((GAT_CH, K), jnp.int32),
      pltpu.VMEM((K,), jnp.int32),
      pltpu.VMEM((K,), jnp.int32),
      pltpu.VMEM((K, HD), jnp.float32),
      pltpu.VMEM((K, HD), jnp.float32),
      pltpu.VMEM((K,), jnp.float32),
      pltpu.VMEM((K,), jnp.float32),
      pltpu.VMEM((ZR, HD), jnp.float32),
      pltpu.VMEM((N,), jnp.float32),
      pltpu.VMEM_SHARED((N, HD), jnp.float32),
      pltpu.SemaphoreType.DMA,
      pltpu.SemaphoreType.DMA,
  )

  def body(h_hbm, src_hbm, dst2_hbm, asd_hbm, o_out, den_out,
           asd_v, dst2d, srcb0, srcb1, rows0, rows1, eeb0, eeb1,
           zbuf, dent, accum, sem0, sem1):
    cid = lax.axis_index("c")
    sid = lax.axis_index("s")
    wid = cid * NS + sid

    pltpu.sync_copy(asd_hbm, asd_v)
    pltpu.sync_copy(dst2_hbm.at[pl.ds(sid * GAT_CH, GAT_CH)], dst2d)
    _fill_zero(zbuf)
    _zero_table(dent)

    col_s = jnp.full((16,), cid, jnp.int32)
    col_d = jnp.full((16,), 2 + cid, jnp.int32)
    iota16 = lax.iota(jnp.int32, 16)

    def mx_body(i, mv):
      v = plsc.load_gather(asd_v, [(iota16 + i * 16) * 4 + col_s])
      return jnp.maximum(mv, v)

    mv = lax.fori_loop(0, N // 16, mx_body,
                       jnp.full((16,), -3.4e38, jnp.float32))
    msv = jnp.full((16,), jnp.max(mv, axis=0), jnp.float32)

    for f in range(2):
      _zero_rows(zbuf, accum, sid)
      plsc.subcore_barrier()
      roff = (cid * 2 + f) * N

      def stage(ch, srcb, eeb, rows, sem):
        pltpu.sync_copy(src_hbm.at[pl.ds(sid * EPC + ch * K, K)], srcb)
        for k in range(K // 16):
          s16 = srcb[pl.ds(k * 16, 16)]
          d16 = dst2d[ch, pl.ds(k * 16, 16)]
          es = plsc.load_gather(asd_v, [s16 * 4 + col_s])
          ad = plsc.load_gather(asd_v, [d16 * 4 + col_d])
          t = es + ad
          e = jnp.where(t > 0, t, 0.2 * t)
          c0 = msv + ad
          cc = jnp.where(c0 > 0, c0, 0.2 * c0)
          ee = jnp.exp(e - cc)
          if f == 0:
            plsc.addupdate_scatter(dent, [d16], ee)
          eeb[pl.ds(k * 16, 16)] = ee
          srcb[pl.ds(k * 16, 16)] = s16 + roff
        pltpu.async_copy(h_hbm.at[srcb], rows, sem)

      def drain(ch, eeb, rows, sem):
        pltpu.make_async_copy(h_hbm.at[pl.ds(0, K)], rows, sem).wait()

        def scale(r, _):
          av = plsc.load_gather(eeb, [jnp.full((16,), r, jnp.int32)])
          for q in range(HD // 16):
            rows[r, pl.ds(q * 16, 16)] = rows[r, pl.ds(q * 16, 16)] * av
          return 0

        lax.fori_loop(0, K, scale, 0)
        pltpu.sync_copy(rows, accum.at[dst2d.at[ch]], add=True)

      stage(0, srcb0, eeb0, rows0, sem0)

      def pair(j, _):
        stage(2 * j + 1, srcb1, eeb1, rows1, sem1)
        drain(2 * j, eeb0, rows0, sem0)

        @pl.when(j < GAT_CH // 2 - 1)
        def _():
          stage(2 * j + 2, srcb0, eeb0, rows0, sem0)

        drain(2 * j + 1, eeb1, rows1, sem1)
        return 0

      lax.fori_loop(0, GAT_CH // 2, pair, 0)
      plsc.subcore_barrier()

      _writeout_rows(accum, o_out, sid, roff)
      if f == 0:
        plsc.subcore_barrier()
    pltpu.sync_copy(dent, den_out.at[pl.ds(wid * N, N)])

  return pl.kernel(body, out_type=out_type, mesh=_mesh,
                   scratch_types=scratch, compiler_params=_sc_params)


_gat = _gat_kernel()


R = 1000
GRID = N // R
_f32 = jnp.float32


def _cat(a, b):
  return jnp.concatenate([a, b], axis=-1)


def _tc1_body(pos, p, degp, wl, wr, b, x1):
  d = jnp.sum(degp[...], axis=0)
  cnt = jnp.maximum(d, 1.0)
  aggr = (p[0] + p[1]) / cnt
  y = (jnp.dot(aggr, wl[...], preferred_element_type=_f32)
       + jnp.dot(pos[...], wr[...], preferred_element_type=_f32) + b[...])
  x1[...] = jnp.maximum(y, 0.0)


def _tc2_body(x1, q, degp, wl, wr, b, wg, ats, atd, h, asd):
  d = jnp.sum(degp[...], axis=0)
  cnt = jnp.maximum(d, 1.0)
  aggr = (q[0] + q[1]) / cnt
  x1v = x1[...]
  x2 = jnp.maximum(
      jnp.dot(aggr, wl[...], preferred_element_type=_f32)
      + jnp.dot(x1v, wr[...], preferred_element_type=_f32) + b[...], 0.0)
  wgv = wg[...]
  hcat = (jnp.dot(x1v, wgv[:D, :], preferred_element_type=_f32)
          + jnp.dot(x2, wgv[D:, :], preferred_element_type=_f32))
  h0 = hcat[:, :D]
  h1 = hcat[:, D:]
  for i in range(4):
    h[i] = hcat[:, i * HD:(i + 1) * HD]
  atsv = ats[...]
  atdv = atd[...]
  asd[:, 0:1] = jnp.sum(h0 * atsv[0:1, :], axis=-1, keepdims=True)
  asd[:, 1:2] = jnp.sum(h1 * atsv[1:2, :], axis=-1, keepdims=True)
  asd[:, 2:3] = jnp.sum(h0 * atdv[0:1, :], axis=-1, keepdims=True)
  asd[:, 3:4] = jnp.sum(h1 * atdv[1:2, :], axis=-1, keepdims=True)


_BN_SCALE = float(1.0 / (1.0 + EPS) ** 0.5)


def _tc3_body(o, denp, bg, batch, wd1, bd1, g1, be1, wd2, bd2, g2, be2,
              wd3, bd3, g3, be3, z, am, gs, gc):
  i = pl.program_id(0)
  den0 = jnp.sum(denp[0], axis=0)
  den1 = jnp.sum(denp[1], axis=0)
  o0 = _cat(o[0], o[1])
  o1 = _cat(o[2], o[3])
  out = 0.5 * (o0 / (den0 + 1e-16) + o1 / (den1 + 1e-16)) + bg[...]
  gid = lax.broadcasted_iota(jnp.int32, (R, B), 1)
  oh = (batch[...] == gid).astype(_f32)
  gs_inc = lax.dot_general(oh, out, (((0,), (0,)), ((), ())),
                           preferred_element_type=_f32)
  gc_inc = lax.dot_general(oh, jnp.ones((R, 1), _f32),
                           (((0,), (0,)), ((), ())),
                           preferred_element_type=_f32)

  @pl.when(i == 0)
  def _():
    gs[...] = jnp.zeros_like(gs)
    gc[...] = jnp.zeros_like(gc)

  gs[...] += gs_inc
  gc[...] += gc_inc

  @pl.when(i == GRID - 1)
  def _():
    zv = gs[...] / jnp.maximum(gc[...], 1.0)

    def bn(x, g, bb):
      return x * _BN_SCALE * g[...] + bb[...]

    y = jnp.maximum(
        bn(jnp.dot(zv, wd1[...], preferred_element_type=_f32) + bd1[...],
           g1, be1), 0.0)
    y = jnp.maximum(
        bn(jnp.dot(y, wd2[...], preferred_element_type=_f32) + bd2[...],
           g2, be2), 0.0)
    y = bn(jnp.dot(y, wd3[...], preferred_element_type=_f32) + bd3[...],
           g3, be3)
    z[...] = y
    mx = jnp.max(y, axis=-1, keepdims=True)
    ii = lax.broadcasted_iota(jnp.int32, (B, OUT), 1)
    am[...] = jnp.min(jnp.where(y >= mx, ii, OUT), axis=-1, keepdims=True)


def _full(shape):
  return pl.BlockSpec(shape, lambda i: tuple(0 for _ in shape))


_tc1 = pl.pallas_call(
    _tc1_body,
    grid=(GRID,),
    in_specs=[
        pl.BlockSpec((R, D), lambda i: (i, 0)),
        pl.BlockSpec((NC, R, D), lambda i: (0, i, 0)),
        pl.BlockSpec((NW, R, 1), lambda i: (0, i, 0)),
        _full((D, D)), _full((D, D)), _full((1, D)),
    ],
    out_specs=pl.BlockSpec((R, D), lambda i: (i, 0)),
    out_shape=jax.ShapeDtypeStruct((N, D), _f32),
)

_tc2 = pl.pallas_call(
    _tc2_body,
    grid=(GRID,),
    in_specs=[
        pl.BlockSpec((R, D), lambda i: (i, 0)),
        pl.BlockSpec((NC, R, D), lambda i: (0, i, 0)),
        pl.BlockSpec((NW, R, 1), lambda i: (0, i, 0)),
        _full((D, D)), _full((D, D)), _full((1, D)),
        _full((2 * D, 2 * D)), _full((HEADS, D)), _full((HEADS, D)),
    ],
    out_specs=[
        pl.BlockSpec((4, R, HD), lambda i: (0, i, 0)),
        pl.BlockSpec((R, 4), lambda i: (i, 0)),
    ],
    out_shape=[
        jax.ShapeDtypeStruct((4, N, HD), _f32),
        jax.ShapeDtypeStruct((N, 4), _f32),
    ],
)

_tc3 = pl.pallas_call(
    _tc3_body,
    grid=(GRID,),
    in_specs=[
        pl.BlockSpec((4, R, HD), lambda i: (0, i, 0)),
        pl.BlockSpec((HEADS, NS, R, 1), lambda i: (0, 0, i, 0)),
        _full((1, D)),
        pl.BlockSpec((R, 1), lambda i: (i, 0)),
        _full((D, 2 * D)), _full((1, 2 * D)), _full((1, 2 * D)),
        _full((1, 2 * D)),
        _full((2 * D, D)), _full((1, D)), _full((1, D)), _full((1, D)),
        _full((D, OUT)), _full((1, OUT)), _full((1, OUT)), _full((1, OUT)),
    ],
    out_specs=[
        pl.BlockSpec((B, OUT), lambda i: (0, 0)),
        pl.BlockSpec((B, 1), lambda i: (0, 0)),
    ],
    out_shape=[
        jax.ShapeDtypeStruct((B, OUT), _f32),
        jax.ShapeDtypeStruct((B, 1), jnp.int32),
    ],
    scratch_shapes=[
        pltpu.VMEM((B, D), _f32),
        pltpu.VMEM((B, 1), _f32),
    ],
    compiler_params=pltpu.CompilerParams(
        dimension_semantics=("arbitrary",)),
)


def kernel(pos, edge_index, batch, W1_l, W1_r, b1, W2_l, W2_r, b2, W_gat,
           att_src, att_dst, b_gat, Wd1, bd1, g1, be1, Wd2, bd2, g2, be2,
           Wd3, bd3, g3, be3):
  src = edge_index[0]
  dst = edge_index[1]
  dst2 = dst.reshape(E // K, K)

  p, degp = _sage_deg(pos, src, dst2)
  p = p.reshape(NC, N, D)
  degp3 = degp.reshape(NW, N, 1)
  x1 = _tc1(pos, p, degp3, W1_l, W1_r, b1.reshape(1, D))

  q = _sage(x1, src, dst2)[0].reshape(NC, N, D)
  h, asd = _tc2(x1, q, degp3, W2_l, W2_r, b2.reshape(1, D),
                W_gat, att_src, att_dst)

  o, denp = _gat(h.reshape(4 * N, HD), src, dst2, asd.reshape(4 * N))
  o = o.reshape(4, N, HD)
  denp4 = denp.reshape(HEADS, NS, N, 1)

  z, am = _tc3(o, denp4, b_gat.reshape(1, D), batch.reshape(N, 1),
               Wd1, bd1.reshape(1, -1), g1.reshape(1, -1), be1.reshape(1, -1),
               Wd2, bd2.reshape(1, -1), g2.reshape(1, -1), be2.reshape(1, -1),
               Wd3, bd3.reshape(1, -1), g3.reshape(1, -1), be3.reshape(1, -1))
  return (z, am.reshape(B))


# speedup vs baseline: 29.3918x; 1.1431x over previous
import jax
import jax.numpy as jnp
from jax import lax
from jax.experimental import pallas as pl
from jax.experimental.pallas import tpu as pltpu
from jax.experimental.pallas import tpu_sc as plsc

N = 10000
E = 320000
D = 128
HD = D // 2
B = 16
HEADS = 2
EPS = 1e-5
OUT = 40

NC = 2
NS = 16
NW = NC * NS

K = 80
EPW = E // NW
EPC = E // NS
GAT_CH = EPC // K
SLAB = 640
SLAB_LAST = N - SLAB * (NS - 1)
ZR = 40

_mesh = plsc.VectorSubcoreMesh(
    core_axis_name="c", subcore_axis_name="s", num_cores=NC, num_subcores=NS)
_sc_params = pltpu.CompilerParams(needs_layout_passes=False,
                                  use_tc_tiling_on_sc=False)


def _fill_zero(zbuf):
  w = zbuf.shape[-1]
  for i in range(ZR):
    for q in range(w // 16):
      zbuf[i, pl.ds(q * 16, 16)] = jnp.zeros((16,), jnp.float32)


def _zero_rows(zbuf, accum, sid):
  @pl.when(sid < NS - 1)
  def _():
    for k in range(SLAB // ZR):
      pltpu.sync_copy(zbuf, accum.at[pl.ds(sid * SLAB + k * ZR, ZR)])

  @pl.when(sid == NS - 1)
  def _():
    for k in range(SLAB_LAST // ZR):
      pltpu.sync_copy(zbuf, accum.at[pl.ds((NS - 1) * SLAB + k * ZR, ZR)])


def _writeout_rows(accum, out_hbm, sid, roff):
  @pl.when(sid < NS - 1)
  def _():
    pltpu.sync_copy(accum.at[pl.ds(sid * SLAB, SLAB)],
                    out_hbm.at[pl.ds(roff + sid * SLAB, SLAB)])

  @pl.when(sid == NS - 1)
  def _():
    pltpu.sync_copy(accum.at[pl.ds((NS - 1) * SLAB, SLAB_LAST)],
                    out_hbm.at[pl.ds(roff + (NS - 1) * SLAB, SLAB_LAST)])


def _zero_table(tab):
  def body(i, _):
    tab[pl.ds(i * 16, 16)] = jnp.zeros((16,), jnp.float32)
    return 0

  lax.fori_loop(0, N // 16, body, 0)


SAGE_CH = EPW // K


def _make_sage(with_deg):
  out_type = [jax.ShapeDtypeStruct((NC * N, D), jnp.float32)]
  if with_deg:
    out_type.append(jax.ShapeDtypeStruct((NW * N,), jnp.float32))
  scratch = [
      pltpu.VMEM((SAGE_CH, K), jnp.int32),
      pltpu.VMEM((K,), jnp.int32),
      pltpu.VMEM((K,), jnp.int32),
      pltpu.VMEM((K, D), jnp.float32),
      pltpu.VMEM((K, D), jnp.float32),
      pltpu.VMEM((ZR, D), jnp.float32),
      pltpu.VMEM_SHARED((N, D), jnp.float32),
      pltpu.SemaphoreType.DMA,
      pltpu.SemaphoreType.DMA,
  ]
  if with_deg:
    scratch.append(pltpu.VMEM((N,), jnp.float32))

  def body(x_hbm, src_hbm, dst2_hbm, p_out, *rest):
    if with_deg:
      (deg_out, dst2d, srcb0, srcb1, rows0, rows1, zbuf, accum,
       sem0, sem1, degt) = rest
    else:
      (dst2d, srcb0, srcb1, rows0, rows1, zbuf, accum, sem0, sem1) = rest
    cid = lax.axis_index("c")
    sid = lax.axis_index("s")
    wid = cid * NS + sid

    pltpu.sync_copy(dst2_hbm.at[pl.ds(wid * SAGE_CH, SAGE_CH)], dst2d)
    _fill_zero(zbuf)
    if with_deg:
      _zero_table(degt)
    ones16 = jnp.ones((16,), jnp.float32)

    _zero_rows(zbuf, accum, sid)
    plsc.subcore_barrier()

    def stage(ch, srcb, rows, sem):
      pltpu.sync_copy(src_hbm.at[pl.ds(wid * EPW + ch * K, K)], srcb)
      pltpu.async_copy(x_hbm.at[srcb], rows, sem)

    def drain(ch, rows, sem):
      pltpu.make_async_copy(x_hbm.at[pl.ds(0, K)], rows, sem).wait()
      pltpu.sync_copy(rows, accum.at[dst2d.at[ch]], add=True)
      if with_deg:
        for k in range(K // 16):
          d16 = dst2d[ch, pl.ds(k * 16, 16)]
          plsc.addupdate_scatter(degt, [d16], ones16)

    stage(0, srcb0, rows0, sem0)

    def pair(j, _):
      stage(2 * j + 1, srcb1, rows1, sem1)
      drain(2 * j, rows0, sem0)
      stage(2 * j + 2, srcb0, rows0, sem0)
      drain(2 * j + 1, rows1, sem1)
      return 0

    lax.fori_loop(0, SAGE_CH // 2, pair, 0)
    drain(SAGE_CH - 1, rows0, sem0)
    plsc.subcore_barrier()

    _writeout_rows(accum, p_out, sid, cid * N)
    if with_deg:
      pltpu.sync_copy(degt, deg_out.at[pl.ds(wid * N, N)])

  return pl.kernel(body, out_type=tuple(out_type), mesh=_mesh,
                   scratch_types=tuple(scratch), compiler_params=_sc_params)


_sage_deg = _make_sage(True)
_sage = _make_sage(False)


def _gat_kernel():
  out_type = (jax.ShapeDtypeStruct((4 * N, HD), jnp.float32),
              jax.ShapeDtypeStruct((NW * N,), jnp.float32))
  scratch = (
      pltpu.VMEM((4 * N,), jnp.float32),
      pltpu.VMEM((GAT_CH, K), jnp.int32),
      pltpu.VMEM((K,), jnp.int32),
      pltpu.VMEM((K,), jnp.int32),
      pltpu.VMEM((K, HD), jnp.float32),
      pltpu.VMEM((K, HD), jnp.float32),
      pltpu.VMEM((K,), jnp.float32),
      pltpu.VMEM((K,), jnp.float32),
      pltpu.VMEM((ZR, HD), jnp.float32),
      pltpu.VMEM((N,), jnp.float32),
      pltpu.VMEM_SHARED((N, HD), jnp.float32),
      pltpu.SemaphoreType.DMA,
      pltpu.SemaphoreType.DMA,
  )

  def body(h_hbm, src_hbm, dst2_hbm, asd_hbm, o_out, den_out,
           asd_v, dst2d, srcb0, srcb1, rows0, rows1, eeb0, eeb1,
           zbuf, dent, accum, sem0, sem1):
    cid = lax.axis_index("c")
    sid = lax.axis_index("s")
    wid = cid * NS + sid

    pltpu.sync_copy(asd_hbm, asd_v)
    pltpu.sync_copy(dst2_hbm.at[pl.ds(sid * GAT_CH, GAT_CH)], dst2d)
    _fill_zero(zbuf)
    _zero_table(dent)

    col_s = jnp.full((16,), cid, jnp.int32)
    col_d = jnp.full((16,), 2 + cid, jnp.int32)
    iota16 = lax.iota(jnp.int32, 16)

    def mx_body(i, mv):
      v = plsc.load_gather(asd_v, [(iota16 + i * 16) * 4 + col_s])
      return jnp.maximum(mv, v)

    mv = lax.fori_loop(0, N // 16, mx_body,
                       jnp.full((16,), -3.4e38, jnp.float32))
    msv = jnp.full((16,), jnp.max(mv, axis=0), jnp.float32)

    for f in range(2):
      _zero_rows(zbuf, accum, sid)
      plsc.subcore_barrier()
      roff = (cid * 2 + f) * N

      def stage(ch, srcb, eeb, rows, sem):
        pltpu.sync_copy(src_hbm.at[pl.ds(sid * EPC + ch * K, K)], srcb)
        for k in range(K // 16):
          s16 = srcb[pl.ds(k * 16, 16)]
          d16 = dst2d[ch, pl.ds(k * 16, 16)]
          es = plsc.load_gather(asd_v, [s16 * 4 + col_s])
          ad = plsc.load_gather(asd_v, [d16 * 4 + col_d])
          t = es + ad
          e = jnp.where(t > 0, t, 0.2 * t)
          c0 = msv + ad
          cc = jnp.where(c0 > 0, c0, 0.2 * c0)
          ee = jnp.exp(e - cc)
          if f == 0:
            plsc.addupdate_scatter(dent, [d16], ee)
          eeb[pl.ds(k * 16, 16)] = ee
          srcb[pl.ds(k * 16, 16)] = s16 + roff
        pltpu.async_copy(h_hbm.at[srcb], rows, sem)

      def drain(ch, eeb, rows, sem):
        pltpu.make_async_copy(h_hbm.at[pl.ds(0, K)], rows, sem).wait()

        @plsc.parallel_loop(0, K, unroll=4)
        def _(r):
          av = plsc.load_gather(eeb, [jnp.full((16,), r, jnp.int32)])
          for q in range(HD // 16):
            rows[r, pl.ds(q * 16, 16)] = rows[r, pl.ds(q * 16, 16)] * av

        pltpu.sync_copy(rows, accum.at[dst2d.at[ch]], add=True)

      stage(0, srcb0, eeb0, rows0, sem0)

      def pair(j, _):
        stage(2 * j + 1, srcb1, eeb1, rows1, sem1)
        drain(2 * j, eeb0, rows0, sem0)

        @pl.when(j < GAT_CH // 2 - 1)
        def _():
          stage(2 * j + 2, srcb0, eeb0, rows0, sem0)

        drain(2 * j + 1, eeb1, rows1, sem1)
        return 0

      lax.fori_loop(0, GAT_CH // 2, pair, 0)
      plsc.subcore_barrier()

      _writeout_rows(accum, o_out, sid, roff)
      if f == 0:
        plsc.subcore_barrier()
    pltpu.sync_copy(dent, den_out.at[pl.ds(wid * N, N)])

  return pl.kernel(body, out_type=out_type, mesh=_mesh,
                   scratch_types=scratch, compiler_params=_sc_params)


_gat = _gat_kernel()


R = 1000
GRID = N // R
_f32 = jnp.float32


def _cat(a, b):
  return jnp.concatenate([a, b], axis=-1)


def _tc1_body(pos, p, degp, wl, wr, b, x1):
  d = jnp.sum(degp[...], axis=0)
  cnt = jnp.maximum(d, 1.0)
  aggr = (p[0] + p[1]) / cnt
  y = (jnp.dot(aggr, wl[...], preferred_element_type=_f32)
       + jnp.dot(pos[...], wr[...], preferred_element_type=_f32) + b[...])
  x1[...] = jnp.maximum(y, 0.0)


def _tc2_body(x1, q, degp, wl, wr, b, wg, ats, atd, h, asd):
  d = jnp.sum(degp[...], axis=0)
  cnt = jnp.maximum(d, 1.0)
  aggr = (q[0] + q[1]) / cnt
  x1v = x1[...]
  x2 = jnp.maximum(
      jnp.dot(aggr, wl[...], preferred_element_type=_f32)
      + jnp.dot(x1v, wr[...], preferred_element_type=_f32) + b[...], 0.0)
  wgv = wg[...]
  hcat = (jnp.dot(x1v, wgv[:D, :], preferred_element_type=_f32)
          + jnp.dot(x2, wgv[D:, :], preferred_element_type=_f32))
  h0 = hcat[:, :D]
  h1 = hcat[:, D:]
  for i in range(4):
    h[i] = hcat[:, i * HD:(i + 1) * HD]
  atsv = ats[...]
  atdv = atd[...]
  asd[:, 0:1] = jnp.sum(h0 * atsv[0:1, :], axis=-1, keepdims=True)
  asd[:, 1:2] = jnp.sum(h1 * atsv[1:2, :], axis=-1, keepdims=True)
  asd[:, 2:3] = jnp.sum(h0 * atdv[0:1, :], axis=-1, keepdims=True)
  asd[:, 3:4] = jnp.sum(h1 * atdv[1:2, :], axis=-1, keepdims=True)


_BN_SCALE = float(1.0 / (1.0 + EPS) ** 0.5)


def _tc3_body(o, denp, bg, batch, wd1, bd1, g1, be1, wd2, bd2, g2, be2,
              wd3, bd3, g3, be3, z, am, gs, gc):
  i = pl.program_id(0)
  den0 = jnp.sum(denp[0], axis=0)
  den1 = jnp.sum(denp[1], axis=0)
  o0 = _cat(o[0], o[1])
  o1 = _cat(o[2], o[3])
  out = 0.5 * (o0 / (den0 + 1e-16) + o1 / (den1 + 1e-16)) + bg[...]
  gid = lax.broadcasted_iota(jnp.int32, (R, B), 1)
  oh = (batch[...] == gid).astype(_f32)
  gs_inc = lax.dot_general(oh, out, (((0,), (0,)), ((), ())),
                           preferred_element_type=_f32)
  gc_inc = lax.dot_general(oh, jnp.ones((R, 1), _f32),
                           (((0,), (0,)), ((), ())),
                           preferred_element_type=_f32)

  @pl.when(i == 0)
  def _():
    gs[...] = jnp.zeros_like(gs)
    gc[...] = jnp.zeros_like(gc)

  gs[...] += gs_inc
  gc[...] += gc_inc

  @pl.when(i == GRID - 1)
  def _():
    zv = gs[...] / jnp.maximum(gc[...], 1.0)

    def bn(x, g, bb):
      return x * _BN_SCALE * g[...] + bb[...]

    y = jnp.maximum(
        bn(jnp.dot(zv, wd1[...], preferred_element_type=_f32) + bd1[...],
           g1, be1), 0.0)
    y = jnp.maximum(
        bn(jnp.dot(y, wd2[...], preferred_element_type=_f32) + bd2[...],
           g2, be2), 0.0)
    y = bn(jnp.dot(y, wd3[...], preferred_element_type=_f32) + bd3[...],
           g3, be3)
    z[...] = y
    mx = jnp.max(y, axis=-1, keepdims=True)
    ii = lax.broadcasted_iota(jnp.int32, (B, OUT), 1)
    am[...] = jnp.min(jnp.where(y >= mx, ii, OUT), axis=-1, keepdims=True)


def _full(shape):
  return pl.BlockSpec(shape, lambda i: tuple(0 for _ in shape))


_tc1 = pl.pallas_call(
    _tc1_body,
    grid=(GRID,),
    in_specs=[
        pl.BlockSpec((R, D), lambda i: (i, 0)),
        pl.BlockSpec((NC, R, D), lambda i: (0, i, 0)),
        pl.BlockSpec((NW, R, 1), lambda i: (0, i, 0)),
        _full((D, D)), _full((D, D)), _full((1, D)),
    ],
    out_specs=pl.BlockSpec((R, D), lambda i: (i, 0)),
    out_shape=jax.ShapeDtypeStruct((N, D), _f32),
)

_tc2 = pl.pallas_call(
    _tc2_body,
    grid=(GRID,),
    in_specs=[
        pl.BlockSpec((R, D), lambda i: (i, 0)),
        pl.BlockSpec((NC, R, D), lambda i: (0, i, 0)),
        pl.BlockSpec((NW, R, 1), lambda i: (0, i, 0)),
        _full((D, D)), _full((D, D)), _full((1, D)),
        _full((2 * D, 2 * D)), _full((HEADS, D)), _full((HEADS, D)),
    ],
    out_specs=[
        pl.BlockSpec((4, R, HD), lambda i: (0, i, 0)),
        pl.BlockSpec((R, 4), lambda i: (i, 0)),
    ],
    out_shape=[
        jax.ShapeDtypeStruct((4, N, HD), _f32),
        jax.ShapeDtypeStruct((N, 4), _f32),
    ],
)

_tc3 = pl.pallas_call(
    _tc3_body,
    grid=(GRID,),
    in_specs=[
        pl.BlockSpec((4, R, HD), lambda i: (0, i, 0)),
        pl.BlockSpec((HEADS, NS, R, 1), lambda i: (0, 0, i, 0)),
        _full((1, D)),
        pl.BlockSpec((R, 1), lambda i: (i, 0)),
        _full((D, 2 * D)), _full((1, 2 * D)), _full((1, 2 * D)),
        _full((1, 2 * D)),
        _full((2 * D, D)), _full((1, D)), _full((1, D)), _full((1, D)),
        _full((D, OUT)), _full((1, OUT)), _full((1, OUT)), _full((1, OUT)),
    ],
    out_specs=[
        pl.BlockSpec((B, OUT), lambda i: (0, 0)),
        pl.BlockSpec((B, 1), lambda i: (0, 0)),
    ],
    out_shape=[
        jax.ShapeDtypeStruct((B, OUT), _f32),
        jax.ShapeDtypeStruct((B, 1), jnp.int32),
    ],
    scratch_shapes=[
        pltpu.VMEM((B, D), _f32),
        pltpu.VMEM((B, 1), _f32),
    ],
    compiler_params=pltpu.CompilerParams(
        dimension_semantics=("arbitrary",)),
)


def kernel(pos, edge_index, batch, W1_l, W1_r, b1, W2_l, W2_r, b2, W_gat,
           att_src, att_dst, b_gat, Wd1, bd1, g1, be1, Wd2, bd2, g2, be2,
           Wd3, bd3, g3, be3):
  src = edge_index[0]
  dst = edge_index[1]
  dst2 = dst.reshape(E // K, K)

  p, degp = _sage_deg(pos, src, dst2)
  p = p.reshape(NC, N, D)
  degp3 = degp.reshape(NW, N, 1)
  x1 = _tc1(pos, p, degp3, W1_l, W1_r, b1.reshape(1, D))

  q = _sage(x1, src, dst2)[0].reshape(NC, N, D)
  h, asd = _tc2(x1, q, degp3, W2_l, W2_r, b2.reshape(1, D),
                W_gat, att_src, att_dst)

  o, denp = _gat(h.reshape(4 * N, HD), src, dst2, asd.reshape(4 * N))
  o = o.reshape(4, N, HD)
  denp4 = denp.reshape(HEADS, NS, N, 1)

  z, am = _tc3(o, denp4, b_gat.reshape(1, D), batch.reshape(N, 1),
               Wd1, bd1.reshape(1, -1), g1.reshape(1, -1), be1.reshape(1, -1),
               Wd2, bd2.reshape(1, -1), g2.reshape(1, -1), be2.reshape(1, -1),
               Wd3, bd3.reshape(1, -1), g3.reshape(1, -1), be3.reshape(1, -1))
  return (z, am.reshape(B))


# speedup vs baseline: 36.3419x; 1.2365x over previous
import jax
import jax.numpy as jnp
from jax import lax
from jax.experimental import pallas as pl
from jax.experimental.pallas import tpu as pltpu
from jax.experimental.pallas import tpu_sc as plsc

N = 10000
E = 320000
D = 128
HD = D // 2
B = 16
HEADS = 2
EPS = 1e-5
OUT = 40

NC = 2
NS = 16
NW = NC * NS

K = 16
EPW = E // NW
EPC = E // NS
SLAB = 640
SLAB_LAST = N - SLAB * (NS - 1)
ZR = 40

_mesh = plsc.VectorSubcoreMesh(
    core_axis_name="c", subcore_axis_name="s", num_cores=NC, num_subcores=NS)
_sc_params = pltpu.CompilerParams(needs_layout_passes=False,
                                  use_tc_tiling_on_sc=False)


def _fill_zero(zbuf):
  w = zbuf.shape[-1]
  for i in range(ZR):
    for q in range(w // 16):
      zbuf[i, pl.ds(q * 16, 16)] = jnp.zeros((16,), jnp.float32)


def _zero_rows(zbuf, accum, sid):
  @pl.when(sid < NS - 1)
  def _():
    for k in range(SLAB // ZR):
      pltpu.sync_copy(zbuf, accum.at[pl.ds(sid * SLAB + k * ZR, ZR)])

  @pl.when(sid == NS - 1)
  def _():
    for k in range(SLAB_LAST // ZR):
      pltpu.sync_copy(zbuf, accum.at[pl.ds((NS - 1) * SLAB + k * ZR, ZR)])


def _writeout_rows(accum, out_hbm, sid, roff):
  @pl.when(sid < NS - 1)
  def _():
    pltpu.sync_copy(accum.at[pl.ds(sid * SLAB, SLAB)],
                    out_hbm.at[pl.ds(roff + sid * SLAB, SLAB)])

  @pl.when(sid == NS - 1)
  def _():
    pltpu.sync_copy(accum.at[pl.ds((NS - 1) * SLAB, SLAB_LAST)],
                    out_hbm.at[pl.ds(roff + (NS - 1) * SLAB, SLAB_LAST)])


def _zero_table(tab):
  def body(i, _):
    tab[pl.ds(i * 16, 16)] = jnp.zeros((16,), jnp.float32)
    return 0

  lax.fori_loop(0, N // 16, body, 0)


SAGE_CH = EPW // K


def _make_sage(with_deg):
  out_type = [jax.ShapeDtypeStruct((NC * N, D), jnp.float32)]
  if with_deg:
    out_type.append(jax.ShapeDtypeStruct((NW * N,), jnp.float32))
  scratch = [
      pltpu.VMEM((EPW,), jnp.int32),
      pltpu.VMEM((SAGE_CH, K), jnp.int32),
      pltpu.VMEM((K, D), jnp.float32),
      pltpu.VMEM((K, D), jnp.float32),
      pltpu.VMEM((ZR, D), jnp.float32),
      pltpu.VMEM_SHARED((N, D), jnp.float32),
      pltpu.SemaphoreType.DMA,
      pltpu.SemaphoreType.DMA,
      pltpu.SemaphoreType.DMA,
      pltpu.SemaphoreType.DMA,
  ]
  if with_deg:
    scratch.append(pltpu.VMEM((N,), jnp.float32))

  def body(x_hbm, src_hbm, dst2_hbm, p_out, *rest):
    if with_deg:
      (deg_out, src_all, dst2d, rows0, rows1, zbuf, accum,
       g0, g1, s0, s1, degt) = rest
    else:
      (src_all, dst2d, rows0, rows1, zbuf, accum, g0, g1, s0, s1) = rest
    cid = lax.axis_index("c")
    sid = lax.axis_index("s")
    wid = cid * NS + sid

    pltpu.sync_copy(src_hbm.at[pl.ds(wid * EPW, EPW)], src_all)
    pltpu.sync_copy(dst2_hbm.at[pl.ds(wid * SAGE_CH, SAGE_CH)], dst2d)
    _fill_zero(zbuf)
    if with_deg:
      _zero_table(degt)
    ones16 = jnp.ones((16,), jnp.float32)

    _zero_rows(zbuf, accum, sid)
    plsc.subcore_barrier()

    def gather(ch, rows, gsem):
      pltpu.async_copy(x_hbm.at[src_all.at[pl.ds(ch * K, K)]], rows, gsem)

    def wait_bytes(sem):
      pltpu.make_async_copy(x_hbm.at[pl.ds(0, K)], rows0, sem).wait()

    def scatter(ch, rows, ssem):
      pltpu.async_copy(rows, accum.at[dst2d.at[ch]], ssem, add=True)
      if with_deg:
        for k in range(K // 16):
          d16 = dst2d[ch, pl.ds(k * 16, 16)]
          plsc.addupdate_scatter(degt, [d16], ones16)

    gather(0, rows0, g0)
    gather(1, rows1, g1)

    def pair(j, _):
      wait_bytes(g0)
      scatter(2 * j, rows0, s0)
      wait_bytes(g1)
      scatter(2 * j + 1, rows1, s1)
      wait_bytes(s0)
      gather(2 * j + 2, rows0, g0)

      @pl.when(j < SAGE_CH // 2 - 1)
      def _():
        wait_bytes(s1)
        gather(2 * j + 3, rows1, g1)

      return 0

    lax.fori_loop(0, SAGE_CH // 2, pair, 0)
    wait_bytes(g0)
    scatter(SAGE_CH - 1, rows0, s0)
    wait_bytes(s0)
    wait_bytes(s1)
    plsc.subcore_barrier()

    _writeout_rows(accum, p_out, sid, cid * N)
    if with_deg:
      pltpu.sync_copy(degt, deg_out.at[pl.ds(wid * N, N)])

  return pl.kernel(body, out_type=tuple(out_type), mesh=_mesh,
                   scratch_types=tuple(scratch), compiler_params=_sc_params)


_sage_deg = _make_sage(True)
_sage = _make_sage(False)


KG = 80
GAT_CHK = EPC // KG


def _gat_kernel():
  out_type = (jax.ShapeDtypeStruct((4 * N, HD), jnp.float32),
              jax.ShapeDtypeStruct((NW * N,), jnp.float32))
  scratch = (
      pltpu.VMEM((N,), jnp.float32),
      pltpu.VMEM((N,), jnp.float32),
      pltpu.VMEM((EPC,), jnp.int32),
      pltpu.VMEM((GAT_CHK, KG), jnp.int32),
      pltpu.VMEM((KG, HD), jnp.float32),
      pltpu.VMEM((KG, HD), jnp.float32),
      pltpu.VMEM((KG,), jnp.float32),
      pltpu.VMEM((KG,), jnp.float32),
      pltpu.VMEM((ZR, HD), jnp.float32),
      pltpu.VMEM((N,), jnp.float32),
      pltpu.VMEM_SHARED((N, HD), jnp.float32),
      pltpu.SemaphoreType.DMA,
      pltpu.SemaphoreType.DMA,
      pltpu.SemaphoreType.DMA,
      pltpu.SemaphoreType.DMA,
  )

  def body(h_hbm, src_hbm, dst2_hbm, asd_hbm, o_out, den_out,
           as_t, ad_t, src_all, dst2d, rows0, rows1, eeb0, eeb1,
           zbuf, dent, accum, g0, g1, s0, s1):
    cid = lax.axis_index("c")
    sid = lax.axis_index("s")
    wid = cid * NS + sid

    pltpu.sync_copy(asd_hbm.at[pl.ds(cid * N, N)], as_t)
    pltpu.sync_copy(asd_hbm.at[pl.ds((2 + cid) * N, N)], ad_t)
    pltpu.sync_copy(src_hbm.at[pl.ds(sid * EPC, EPC)], src_all)
    pltpu.sync_copy(dst2_hbm.at[pl.ds(sid * GAT_CHK, GAT_CHK)], dst2d)
    _fill_zero(zbuf)
    _zero_table(dent)

    def mx_body(i, mv):
      return jnp.maximum(mv, as_t[pl.ds(i * 16, 16)])

    mv = lax.fori_loop(0, N // 16, mx_body,
                       jnp.full((16,), -3.4e38, jnp.float32))
    msv = jnp.full((16,), jnp.max(mv, axis=0), jnp.float32)

    def adjust_src(delta):
      dv = jnp.full((16,), delta, jnp.int32)

      @plsc.parallel_loop(0, EPC // 16, unroll=4)
      def _(i):
        src_all[pl.ds(i * 16, 16)] = src_all[pl.ds(i * 16, 16)] + dv

    for f in range(2):
      _zero_rows(zbuf, accum, sid)
      adjust_src(cid * 2 * N if f == 0 else N)
      plsc.subcore_barrier()
      roff = (cid * 2 + f) * N
      off16 = jnp.full((16,), roff, jnp.int32)

      def scal(ch, eeb):
        for k in range(KG // 16):
          s16 = src_all[pl.ds(ch * KG + k * 16, 16)] - off16
          d16 = dst2d[ch, pl.ds(k * 16, 16)]
          es = plsc.load_gather(as_t, [s16])
          ad = plsc.load_gather(ad_t, [d16])
          t = es + ad
          e = jnp.where(t > 0, t, 0.2 * t)
          c0 = msv + ad
          cc = jnp.where(c0 > 0, c0, 0.2 * c0)
          ee = jnp.exp(e - cc)
          if f == 0:
            plsc.addupdate_scatter(dent, [d16], ee)
          eeb[pl.ds(k * 16, 16)] = ee

      def gather(ch, rows, gsem):
        pltpu.async_copy(h_hbm.at[src_all.at[pl.ds(ch * KG, KG)]],
                         rows, gsem)

      def wait_bytes(sem):
        pltpu.make_async_copy(h_hbm.at[pl.ds(0, KG)], rows0, sem).wait()

      def scale_scatter(ch, eeb, rows, ssem):
        @plsc.parallel_loop(0, KG, unroll=4)
        def _(r):
          av = plsc.load_gather(eeb, [jnp.full((16,), r, jnp.int32)])
          for q in range(HD // 16):
            rows[r, pl.ds(q * 16, 16)] = rows[r, pl.ds(q * 16, 16)] * av

        pltpu.async_copy(rows, accum.at[dst2d.at[ch]], ssem, add=True)

      scal(0, eeb0)
      gather(0, rows0, g0)
      scal(1, eeb1)
      gather(1, rows1, g1)

      def pair(j, _):
        wait_bytes(g0)
        scale_scatter(2 * j, eeb0, rows0, s0)
        wait_bytes(g1)
        scale_scatter(2 * j + 1, eeb1, rows1, s1)

        @pl.when(j < GAT_CHK // 2 - 1)
        def _():
          scal(2 * j + 2, eeb0)
          wait_bytes(s0)
          gather(2 * j + 2, rows0, g0)
          scal(2 * j + 3, eeb1)
          wait_bytes(s1)
          gather(2 * j + 3, rows1, g1)

        return 0

      lax.fori_loop(0, GAT_CHK // 2, pair, 0)
      wait_bytes(s0)
      wait_bytes(s1)
      plsc.subcore_barrier()

      _writeout_rows(accum, o_out, sid, roff)
      if f == 0:
        plsc.subcore_barrier()
    pltpu.sync_copy(dent, den_out.at[pl.ds(wid * N, N)])

  return pl.kernel(body, out_type=out_type, mesh=_mesh,
                   scratch_types=scratch, compiler_params=_sc_params)


_gat = _gat_kernel()


R = 1000
GRID = N // R
_f32 = jnp.float32


def _cat(a, b):
  return jnp.concatenate([a, b], axis=-1)


def _tc1_body(pos, p, degp, wl, wr, b, x1):
  cnt = jnp.maximum(jnp.sum(degp[...], axis=1, keepdims=True), 1.0)
  aggr = (p[0] + p[1]) / cnt
  y = (jnp.dot(aggr, wl[...], preferred_element_type=_f32)
       + jnp.dot(pos[...], wr[...], preferred_element_type=_f32) + b[...])
  x1[...] = jnp.maximum(y, 0.0)


def _tc2_body(x1, q, degp, wl, wr, b, wg, ats, atd, h, asd):
  cnt = jnp.maximum(jnp.sum(degp[...], axis=1, keepdims=True), 1.0)
  aggr = (q[0] + q[1]) / cnt
  x1v = x1[...]
  x2 = jnp.maximum(
      jnp.dot(aggr, wl[...], preferred_element_type=_f32)
      + jnp.dot(x1v, wr[...], preferred_element_type=_f32) + b[...], 0.0)
  wgv = wg[...]
  hcat = (jnp.dot(x1v, wgv[:D, :], preferred_element_type=_f32)
          + jnp.dot(x2, wgv[D:, :], preferred_element_type=_f32))
  h0 = hcat[:, :D]
  h1 = hcat[:, D:]
  for i in range(4):
    h[i] = hcat[:, i * HD:(i + 1) * HD]
  atsv = ats[...]
  atdv = atd[...]
  asd[:, 0:1] = jnp.sum(h0 * atsv[0:1, :], axis=-1, keepdims=True)
  asd[:, 1:2] = jnp.sum(h1 * atsv[1:2, :], axis=-1, keepdims=True)
  asd[:, 2:3] = jnp.sum(h0 * atdv[0:1, :], axis=-1, keepdims=True)
  asd[:, 3:4] = jnp.sum(h1 * atdv[1:2, :], axis=-1, keepdims=True)


_BN_SCALE = float(1.0 / (1.0 + EPS) ** 0.5)


def _tc3_body(o, denp, bg, batch, wd1, bd1, g1, be1, wd2, bd2, g2, be2,
              wd3, bd3, g3, be3, z, am, gs, gc):
  i = pl.program_id(0)
  dv = denp[...]
  den0 = jnp.sum(dv[:, :NS], axis=1, keepdims=True)
  den1 = jnp.sum(dv[:, NS:], axis=1, keepdims=True)
  o0 = _cat(o[0], o[1])
  o1 = _cat(o[2], o[3])
  out = 0.5 * (o0 / (den0 + 1e-16) + o1 / (den1 + 1e-16)) + bg[...]
  gid = lax.broadcasted_iota(jnp.int32, (R, B), 1)
  oh = (batch[...] == gid).astype(_f32)
  gs_inc = lax.dot_general(oh, out, (((0,), (0,)), ((), ())),
                           preferred_element_type=_f32)
  gc_inc = lax.dot_general(oh, jnp.ones((R, 1), _f32),
                           (((0,), (0,)), ((), ())),
                           preferred_element_type=_f32)

  @pl.when(i == 0)
  def _():
    gs[...] = jnp.zeros_like(gs)
    gc[...] = jnp.zeros_like(gc)

  gs[...] += gs_inc
  gc[...] += gc_inc

  @pl.when(i == GRID - 1)
  def _():
    zv = gs[...] / jnp.maximum(gc[...], 1.0)

    def bn(x, g, bb):
      return x * _BN_SCALE * g[...] + bb[...]

    y = jnp.maximum(
        bn(jnp.dot(zv, wd1[...], preferred_element_type=_f32) + bd1[...],
           g1, be1), 0.0)
    y = jnp.maximum(
        bn(jnp.dot(y, wd2[...], preferred_element_type=_f32) + bd2[...],
           g2, be2), 0.0)
    y = bn(jnp.dot(y, wd3[...], preferred_element_type=_f32) + bd3[...],
           g3, be3)
    z[...] = y
    mx = jnp.max(y, axis=-1, keepdims=True)
    ii = lax.broadcasted_iota(jnp.int32, (B, OUT), 1)
    am[...] = jnp.min(jnp.where(y >= mx, ii, OUT), axis=-1, keepdims=True)


def _full(shape):
  return pl.BlockSpec(shape, lambda i: tuple(0 for _ in shape))


_tc1 = pl.pallas_call(
    _tc1_body,
    grid=(GRID,),
    in_specs=[
        pl.BlockSpec((R, D), lambda i: (i, 0)),
        pl.BlockSpec((NC, R, D), lambda i: (0, i, 0)),
        pl.BlockSpec((R, NW), lambda i: (i, 0)),
        _full((D, D)), _full((D, D)), _full((1, D)),
    ],
    out_specs=pl.BlockSpec((R, D), lambda i: (i, 0)),
    out_shape=jax.ShapeDtypeStruct((N, D), _f32),
)

_tc2 = pl.pallas_call(
    _tc2_body,
    grid=(GRID,),
    in_specs=[
        pl.BlockSpec((R, D), lambda i: (i, 0)),
        pl.BlockSpec((NC, R, D), lambda i: (0, i, 0)),
        pl.BlockSpec((R, NW), lambda i: (i, 0)),
        _full((D, D)), _full((D, D)), _full((1, D)),
        _full((2 * D, 2 * D)), _full((HEADS, D)), _full((HEADS, D)),
    ],
    out_specs=[
        pl.BlockSpec((4, R, HD), lambda i: (0, i, 0)),
        pl.BlockSpec((R, 4), lambda i: (i, 0)),
    ],
    out_shape=[
        jax.ShapeDtypeStruct((4, N, HD), _f32),
        jax.ShapeDtypeStruct((N, 4), _f32),
    ],
)

_tc3 = pl.pallas_call(
    _tc3_body,
    grid=(GRID,),
    in_specs=[
        pl.BlockSpec((4, R, HD), lambda i: (0, i, 0)),
        pl.BlockSpec((R, NW), lambda i: (i, 0)),
        _full((1, D)),
        pl.BlockSpec((R, 1), lambda i: (i, 0)),
        _full((D, 2 * D)), _full((1, 2 * D)), _full((1, 2 * D)),
        _full((1, 2 * D)),
        _full((2 * D, D)), _full((1, D)), _full((1, D)), _full((1, D)),
        _full((D, OUT)), _full((1, OUT)), _full((1, OUT)), _full((1, OUT)),
    ],
    out_specs=[
        pl.BlockSpec((B, OUT), lambda i: (0, 0)),
        pl.BlockSpec((B, 1), lambda i: (0, 0)),
    ],
    out_shape=[
        jax.ShapeDtypeStruct((B, OUT), _f32),
        jax.ShapeDtypeStruct((B, 1), jnp.int32),
    ],
    scratch_shapes=[
        pltpu.VMEM((B, D), _f32),
        pltpu.VMEM((B, 1), _f32),
    ],
    compiler_params=pltpu.CompilerParams(
        dimension_semantics=("arbitrary",)),
)


def kernel(pos, edge_index, batch, W1_l, W1_r, b1, W2_l, W2_r, b2, W_gat,
           att_src, att_dst, b_gat, Wd1, bd1, g1, be1, Wd2, bd2, g2, be2,
           Wd3, bd3, g3, be3):
  src = edge_index[0]
  dst = edge_index[1]
  dst2s = dst.reshape(E // K, K)
  dst2g = dst.reshape(E // KG, KG)

  p, degp = _sage_deg(pos, src, dst2s)
  p = p.reshape(NC, N, D)
  degpT = jnp.transpose(degp.reshape(NW, N))
  x1 = _tc1(pos, p, degpT, W1_l, W1_r, b1.reshape(1, D))

  q = _sage(x1, src, dst2s)[0].reshape(NC, N, D)
  h, asd = _tc2(x1, q, degpT, W2_l, W2_r, b2.reshape(1, D),
                W_gat, att_src, att_dst)

  asdT = jnp.transpose(asd).reshape(4 * N)
  o, denp = _gat(h.reshape(4 * N, HD), src, dst2g, asdT)
  o = o.reshape(4, N, HD)
  denpT = jnp.transpose(denp.reshape(NW, N))

  z, am = _tc3(o, denpT, b_gat.reshape(1, D), batch.reshape(N, 1),
               Wd1, bd1.reshape(1, -1), g1.reshape(1, -1), be1.reshape(1, -1),
               Wd2, bd2.reshape(1, -1), g2.reshape(1, -1), be2.reshape(1, -1),
               Wd3, bd3.reshape(1, -1), g3.reshape(1, -1), be3.reshape(1, -1))
  return (z, am.reshape(B))


# speedup vs baseline: 47.8558x; 1.3168x over previous
import jax
import jax.numpy as jnp
from jax import lax
from jax.experimental import pallas as pl
from jax.experimental.pallas import tpu as pltpu
from jax.experimental.pallas import tpu_sc as plsc

N = 10000
E = 320000
D = 128
HD = D // 2
B = 16
HEADS = 2
EPS = 1e-5
OUT = 40

NC = 2
NS = 16
NW = NC * NS

K = 80
EPW = E // NW
EPC = E // NS
SLAB = 640
SLAB_LAST = N - SLAB * (NS - 1)
ZR = 40

_mesh = plsc.VectorSubcoreMesh(
    core_axis_name="c", subcore_axis_name="s", num_cores=NC, num_subcores=NS)
_sc_params = pltpu.CompilerParams(needs_layout_passes=False,
                                  use_tc_tiling_on_sc=False)


def _fill_zero(zbuf):
  w = zbuf.shape[-1]
  for i in range(ZR):
    for q in range(w // 16):
      zbuf[i, pl.ds(q * 16, 16)] = jnp.zeros((16,), jnp.float32)


def _zero_rows(zbuf, accum, sid):
  zr = zbuf.shape[0]

  @pl.when(sid < NS - 1)
  def _():
    for k in range(SLAB // zr):
      pltpu.sync_copy(zbuf, accum.at[pl.ds(sid * SLAB + k * zr, zr)])

  @pl.when(sid == NS - 1)
  def _():
    for k in range(SLAB_LAST // zr):
      pltpu.sync_copy(zbuf, accum.at[pl.ds((NS - 1) * SLAB + k * zr, zr)])


def _writeout_rows(accum, out_hbm, sid, roff):
  @pl.when(sid < NS - 1)
  def _():
    pltpu.sync_copy(accum.at[pl.ds(sid * SLAB, SLAB)],
                    out_hbm.at[pl.ds(roff + sid * SLAB, SLAB)])

  @pl.when(sid == NS - 1)
  def _():
    pltpu.sync_copy(accum.at[pl.ds((NS - 1) * SLAB, SLAB_LAST)],
                    out_hbm.at[pl.ds(roff + (NS - 1) * SLAB, SLAB_LAST)])


def _zero_table(tab):
  def body(i, _):
    tab[pl.ds(i * 16, 16)] = jnp.zeros((16,), jnp.float32)
    return 0

  lax.fori_loop(0, N // 16, body, 0)


SAGE_CH = EPW // K


def _make_sage(with_deg):
  out_type = [jax.ShapeDtypeStruct((NC * N, D), jnp.float32)]
  if with_deg:
    out_type.append(jax.ShapeDtypeStruct((NW * N,), jnp.float32))
  scratch = [
      pltpu.VMEM((EPW,), jnp.int32),
      pltpu.VMEM((SAGE_CH, K), jnp.int32),
      pltpu.VMEM((K, D), jnp.float32),
      pltpu.VMEM((K, D), jnp.float32),
      pltpu.VMEM_SHARED((N, D), jnp.float32),
      pltpu.SemaphoreType.DMA,
      pltpu.SemaphoreType.DMA,
      pltpu.SemaphoreType.DMA,
      pltpu.SemaphoreType.DMA,
  ]
  if with_deg:
    scratch.append(pltpu.VMEM((N,), jnp.float32))

  def body(x_hbm, src_hbm, dst2_hbm, p_out, *rest):
    if with_deg:
      (deg_out, src_all, dst2d, rows0, rows1, accum,
       g0, g1, s0, s1, degt) = rest
    else:
      (src_all, dst2d, rows0, rows1, accum, g0, g1, s0, s1) = rest
    cid = lax.axis_index("c")
    sid = lax.axis_index("s")
    wid = cid * NS + sid

    pltpu.sync_copy(src_hbm.at[pl.ds(wid * EPW, EPW)], src_all)
    pltpu.sync_copy(dst2_hbm.at[pl.ds(wid * SAGE_CH, SAGE_CH)], dst2d)
    if with_deg:
      _zero_table(degt)
    ones16 = jnp.ones((16,), jnp.float32)

    def zrow(r, _):
      for q in range(D // 16):
        rows0[r, pl.ds(q * 16, 16)] = jnp.zeros((16,), jnp.float32)
      return 0

    lax.fori_loop(0, K, zrow, 0)
    _zero_rows(rows0, accum, sid)
    plsc.subcore_barrier()

    def gather(ch, rows, gsem):
      pltpu.async_copy(x_hbm.at[src_all.at[pl.ds(ch * K, K)]], rows, gsem)

    def wait_bytes(sem):
      pltpu.make_async_copy(x_hbm.at[pl.ds(0, K)], rows0, sem).wait()

    def scatter(ch, rows, ssem):
      pltpu.async_copy(rows, accum.at[dst2d.at[ch]], ssem, add=True)
      if with_deg:
        for k in range(K // 16):
          d16 = dst2d[ch, pl.ds(k * 16, 16)]
          plsc.addupdate_scatter(degt, [d16], ones16)

    gather(0, rows0, g0)
    gather(1, rows1, g1)

    def pair(j, _):
      wait_bytes(g0)
      scatter(2 * j, rows0, s0)
      wait_bytes(g1)
      scatter(2 * j + 1, rows1, s1)
      wait_bytes(s0)
      gather(2 * j + 2, rows0, g0)

      @pl.when(j < SAGE_CH // 2 - 1)
      def _():
        wait_bytes(s1)
        gather(2 * j + 3, rows1, g1)

      return 0

    lax.fori_loop(0, SAGE_CH // 2, pair, 0)
    wait_bytes(g0)
    scatter(SAGE_CH - 1, rows0, s0)
    wait_bytes(s0)
    wait_bytes(s1)
    plsc.subcore_barrier()

    _writeout_rows(accum, p_out, sid, cid * N)
    if with_deg:
      pltpu.sync_copy(degt, deg_out.at[pl.ds(wid * N, N)])

  return pl.kernel(body, out_type=tuple(out_type), mesh=_mesh,
                   scratch_types=tuple(scratch), compiler_params=_sc_params)


_sage_deg = _make_sage(True)
_sage = _make_sage(False)


KG = 80
GAT_CHK = EPC // KG


def _gat_kernel():
  out_type = (jax.ShapeDtypeStruct((4 * N, HD), jnp.float32),
              jax.ShapeDtypeStruct((NW * N,), jnp.float32))
  scratch = (
      pltpu.VMEM((N,), jnp.float32),
      pltpu.VMEM((N,), jnp.float32),
      pltpu.VMEM((EPC,), jnp.int32),
      pltpu.VMEM((GAT_CHK, KG), jnp.int32),
      pltpu.VMEM((KG, HD), jnp.float32),
      pltpu.VMEM((KG, HD), jnp.float32),
      pltpu.VMEM((KG,), jnp.float32),
      pltpu.VMEM((KG,), jnp.float32),
      pltpu.VMEM((ZR, HD), jnp.float32),
      pltpu.VMEM((N,), jnp.float32),
      pltpu.VMEM_SHARED((N, HD), jnp.float32),
      pltpu.SemaphoreType.DMA,
      pltpu.SemaphoreType.DMA,
      pltpu.SemaphoreType.DMA,
      pltpu.SemaphoreType.DMA,
  )

  def body(h_hbm, src_hbm, dst2_hbm, asd_hbm, o_out, den_out,
           as_t, ad_t, src_all, dst2d, rows0, rows1, eeb0, eeb1,
           zbuf, dent, accum, g0, g1, s0, s1):
    cid = lax.axis_index("c")
    sid = lax.axis_index("s")
    wid = cid * NS + sid

    pltpu.sync_copy(asd_hbm.at[pl.ds(cid * N, N)], as_t)
    pltpu.sync_copy(asd_hbm.at[pl.ds((2 + cid) * N, N)], ad_t)
    pltpu.sync_copy(src_hbm.at[pl.ds(sid * EPC, EPC)], src_all)
    pltpu.sync_copy(dst2_hbm.at[pl.ds(sid * GAT_CHK, GAT_CHK)], dst2d)
    _fill_zero(zbuf)
    _zero_table(dent)

    def mx_body(i, mv):
      return jnp.maximum(mv, as_t[pl.ds(i * 16, 16)])

    mv = lax.fori_loop(0, N // 16, mx_body,
                       jnp.full((16,), -3.4e38, jnp.float32))
    msv = jnp.full((16,), jnp.max(mv, axis=0), jnp.float32)

    def adjust_src(delta):
      dv = jnp.full((16,), delta, jnp.int32)

      @plsc.parallel_loop(0, EPC // 16, unroll=4)
      def _(i):
        src_all[pl.ds(i * 16, 16)] = src_all[pl.ds(i * 16, 16)] + dv

    for f in range(2):
      _zero_rows(zbuf, accum, sid)
      adjust_src(cid * 2 * N if f == 0 else N)
      plsc.subcore_barrier()
      roff = (cid * 2 + f) * N
      off16 = jnp.full((16,), roff, jnp.int32)

      def scal(ch, eeb):
        for k in range(KG // 16):
          s16 = src_all[pl.ds(ch * KG + k * 16, 16)] - off16
          d16 = dst2d[ch, pl.ds(k * 16, 16)]
          es = plsc.load_gather(as_t, [s16])
          ad = plsc.load_gather(ad_t, [d16])
          t = es + ad
          e = jnp.where(t > 0, t, 0.2 * t)
          c0 = msv + ad
          cc = jnp.where(c0 > 0, c0, 0.2 * c0)
          ee = jnp.exp(e - cc)
          if f == 0:
            plsc.addupdate_scatter(dent, [d16], ee)
          eeb[pl.ds(k * 16, 16)] = ee

      def gather(ch, rows, gsem):
        pltpu.async_copy(h_hbm.at[src_all.at[pl.ds(ch * KG, KG)]],
                         rows, gsem)

      def wait_bytes(sem):
        pltpu.make_async_copy(h_hbm.at[pl.ds(0, KG)], rows0, sem).wait()

      def scale_scatter(ch, eeb, rows, ssem):
        @plsc.parallel_loop(0, KG, unroll=4)
        def _(r):
          av = plsc.load_gather(eeb, [jnp.full((16,), r, jnp.int32)])
          for q in range(HD // 16):
            rows[r, pl.ds(q * 16, 16)] = rows[r, pl.ds(q * 16, 16)] * av

        pltpu.async_copy(rows, accum.at[dst2d.at[ch]], ssem, add=True)

      scal(0, eeb0)
      gather(0, rows0, g0)
      scal(1, eeb1)
      gather(1, rows1, g1)

      def pair(j, _):
        wait_bytes(g0)
        scale_scatter(2 * j, eeb0, rows0, s0)
        wait_bytes(g1)
        scale_scatter(2 * j + 1, eeb1, rows1, s1)

        @pl.when(j < GAT_CHK // 2 - 1)
        def _():
          scal(2 * j + 2, eeb0)
          wait_bytes(s0)
          gather(2 * j + 2, rows0, g0)
          scal(2 * j + 3, eeb1)
          wait_bytes(s1)
          gather(2 * j + 3, rows1, g1)

        return 0

      lax.fori_loop(0, GAT_CHK // 2, pair, 0)
      wait_bytes(s0)
      wait_bytes(s1)
      plsc.subcore_barrier()

      _writeout_rows(accum, o_out, sid, roff)
      if f == 0:
        plsc.subcore_barrier()
    pltpu.sync_copy(dent, den_out.at[pl.ds(wid * N, N)])

  return pl.kernel(body, out_type=out_type, mesh=_mesh,
                   scratch_types=scratch, compiler_params=_sc_params)


_gat = _gat_kernel()


R = 1000
GRID = N // R
_f32 = jnp.float32


def _cat(a, b):
  return jnp.concatenate([a, b], axis=-1)


def _tc1_body(pos, p, degp, wl, wr, b, x1):
  cnt = jnp.maximum(jnp.sum(degp[...], axis=1, keepdims=True), 1.0)
  aggr = (p[0] + p[1]) / cnt
  y = (jnp.dot(aggr, wl[...], preferred_element_type=_f32)
       + jnp.dot(pos[...], wr[...], preferred_element_type=_f32) + b[...])
  x1[...] = jnp.maximum(y, 0.0)


def _tc2_body(x1, q, degp, wl, wr, b, wg, ats, atd, h, asd):
  cnt = jnp.maximum(jnp.sum(degp[...], axis=1, keepdims=True), 1.0)
  aggr = (q[0] + q[1]) / cnt
  x1v = x1[...]
  x2 = jnp.maximum(
      jnp.dot(aggr, wl[...], preferred_element_type=_f32)
      + jnp.dot(x1v, wr[...], preferred_element_type=_f32) + b[...], 0.0)
  wgv = wg[...]
  hcat = (jnp.dot(x1v, wgv[:D, :], preferred_element_type=_f32)
          + jnp.dot(x2, wgv[D:, :], preferred_element_type=_f32))
  h0 = hcat[:, :D]
  h1 = hcat[:, D:]
  for i in range(4):
    h[i] = hcat[:, i * HD:(i + 1) * HD]
  atsv = ats[...]
  atdv = atd[...]
  asd[:, 0:1] = jnp.sum(h0 * atsv[0:1, :], axis=-1, keepdims=True)
  asd[:, 1:2] = jnp.sum(h1 * atsv[1:2, :], axis=-1, keepdims=True)
  asd[:, 2:3] = jnp.sum(h0 * atdv[0:1, :], axis=-1, keepdims=True)
  asd[:, 3:4] = jnp.sum(h1 * atdv[1:2, :], axis=-1, keepdims=True)


_BN_SCALE = float(1.0 / (1.0 + EPS) ** 0.5)


def _tc3_body(o, denp, bg, batch, wd1, bd1, g1, be1, wd2, bd2, g2, be2,
              wd3, bd3, g3, be3, z, am, gs, gc):
  i = pl.program_id(0)
  dv = denp[...]
  den0 = jnp.sum(dv[:, :NS], axis=1, keepdims=True)
  den1 = jnp.sum(dv[:, NS:], axis=1, keepdims=True)
  o0 = _cat(o[0], o[1])
  o1 = _cat(o[2], o[3])
  out = 0.5 * (o0 / (den0 + 1e-16) + o1 / (den1 + 1e-16)) + bg[...]
  gid = lax.broadcasted_iota(jnp.int32, (R, B), 1)
  oh = (batch[...] == gid).astype(_f32)
  gs_inc = lax.dot_general(oh, out, (((0,), (0,)), ((), ())),
                           preferred_element_type=_f32)
  gc_inc = lax.dot_general(oh, jnp.ones((R, 1), _f32),
                           (((0,), (0,)), ((), ())),
                           preferred_element_type=_f32)

  @pl.when(i == 0)
  def _():
    gs[...] = jnp.zeros_like(gs)
    gc[...] = jnp.zeros_like(gc)

  gs[...] += gs_inc
  gc[...] += gc_inc

  @pl.when(i == GRID - 1)
  def _():
    zv = gs[...] / jnp.maximum(gc[...], 1.0)

    def bn(x, g, bb):
      return x * _BN_SCALE * g[...] + bb[...]

    y = jnp.maximum(
        bn(jnp.dot(zv, wd1[...], preferred_element_type=_f32) + bd1[...],
           g1, be1), 0.0)
    y = jnp.maximum(
        bn(jnp.dot(y, wd2[...], preferred_element_type=_f32) + bd2[...],
           g2, be2), 0.0)
    y = bn(jnp.dot(y, wd3[...], preferred_element_type=_f32) + bd3[...],
           g3, be3)
    z[...] = y
    mx = jnp.max(y, axis=-1, keepdims=True)
    ii = lax.broadcasted_iota(jnp.int32, (B, OUT), 1)
    am[...] = jnp.min(jnp.where(y >= mx, ii, OUT), axis=-1, keepdims=True)


def _full(shape):
  return pl.BlockSpec(shape, lambda i: tuple(0 for _ in shape))


_tc1 = pl.pallas_call(
    _tc1_body,
    grid=(GRID,),
    in_specs=[
        pl.BlockSpec((R, D), lambda i: (i, 0)),
        pl.BlockSpec((NC, R, D), lambda i: (0, i, 0)),
        pl.BlockSpec((R, NW), lambda i: (i, 0)),
        _full((D, D)), _full((D, D)), _full((1, D)),
    ],
    out_specs=pl.BlockSpec((R, D), lambda i: (i, 0)),
    out_shape=jax.ShapeDtypeStruct((N, D), _f32),
)

_tc2 = pl.pallas_call(
    _tc2_body,
    grid=(GRID,),
    in_specs=[
        pl.BlockSpec((R, D), lambda i: (i, 0)),
        pl.BlockSpec((NC, R, D), lambda i: (0, i, 0)),
        pl.BlockSpec((R, NW), lambda i: (i, 0)),
        _full((D, D)), _full((D, D)), _full((1, D)),
        _full((2 * D, 2 * D)), _full((HEADS, D)), _full((HEADS, D)),
    ],
    out_specs=[
        pl.BlockSpec((4, R, HD), lambda i: (0, i, 0)),
        pl.BlockSpec((R, 4), lambda i: (i, 0)),
    ],
    out_shape=[
        jax.ShapeDtypeStruct((4, N, HD), _f32),
        jax.ShapeDtypeStruct((N, 4), _f32),
    ],
)

_tc3 = pl.pallas_call(
    _tc3_body,
    grid=(GRID,),
    in_specs=[
        pl.BlockSpec((4, R, HD), lambda i: (0, i, 0)),
        pl.BlockSpec((R, NW), lambda i: (i, 0)),
        _full((1, D)),
        pl.BlockSpec((R, 1), lambda i: (i, 0)),
        _full((D, 2 * D)), _full((1, 2 * D)), _full((1, 2 * D)),
        _full((1, 2 * D)),
        _full((2 * D, D)), _full((1, D)), _full((1, D)), _full((1, D)),
        _full((D, OUT)), _full((1, OUT)), _full((1, OUT)), _full((1, OUT)),
    ],
    out_specs=[
        pl.BlockSpec((B, OUT), lambda i: (0, 0)),
        pl.BlockSpec((B, 1), lambda i: (0, 0)),
    ],
    out_shape=[
        jax.ShapeDtypeStruct((B, OUT), _f32),
        jax.ShapeDtypeStruct((B, 1), jnp.int32),
    ],
    scratch_shapes=[
        pltpu.VMEM((B, D), _f32),
        pltpu.VMEM((B, 1), _f32),
    ],
    compiler_params=pltpu.CompilerParams(
        dimension_semantics=("arbitrary",)),
)


def kernel(pos, edge_index, batch, W1_l, W1_r, b1, W2_l, W2_r, b2, W_gat,
           att_src, att_dst, b_gat, Wd1, bd1, g1, be1, Wd2, bd2, g2, be2,
           Wd3, bd3, g3, be3):
  src = edge_index[0]
  dst = edge_index[1]
  dst2s = dst.reshape(E // K, K)
  dst2g = dst.reshape(E // KG, KG)

  p, degp = _sage_deg(pos, src, dst2s)
  p = p.reshape(NC, N, D)
  degpT = jnp.transpose(degp.reshape(NW, N))
  x1 = _tc1(pos, p, degpT, W1_l, W1_r, b1.reshape(1, D))

  q = _sage(x1, src, dst2s)[0].reshape(NC, N, D)
  h, asd = _tc2(x1, q, degpT, W2_l, W2_r, b2.reshape(1, D),
                W_gat, att_src, att_dst)

  asdT = jnp.transpose(asd).reshape(4 * N)
  o, denp = _gat(h.reshape(4 * N, HD), src, dst2g, asdT)
  o = o.reshape(4, N, HD)
  denpT = jnp.transpose(denp.reshape(NW, N))

  z, am = _tc3(o, denpT, b_gat.reshape(1, D), batch.reshape(N, 1),
               Wd1, bd1.reshape(1, -1), g1.reshape(1, -1), be1.reshape(1, -1),
               Wd2, bd2.reshape(1, -1), g2.reshape(1, -1), be2.reshape(1, -1),
               Wd3, bd3.reshape(1, -1), g3.reshape(1, -1), be3.reshape(1, -1))
  return (z, am.reshape(B))


# speedup vs baseline: 47.8623x; 1.0001x over previous
import jax
import jax.numpy as jnp
from jax import lax
from jax.experimental import pallas as pl
from jax.experimental.pallas import tpu as pltpu
from jax.experimental.pallas import tpu_sc as plsc

N = 10000
E = 320000
D = 128
HD = D // 2
B = 16
HEADS = 2
EPS = 1e-5
OUT = 40

NC = 2
NS = 16
NW = NC * NS

K = 80
EPW = E // NW
EPC = E // NS
SLAB = 640
SLAB_LAST = N - SLAB * (NS - 1)
ZR = 40

_mesh = plsc.VectorSubcoreMesh(
    core_axis_name="c", subcore_axis_name="s", num_cores=NC, num_subcores=NS)
_sc_params = pltpu.CompilerParams(needs_layout_passes=False,
                                  use_tc_tiling_on_sc=False)


def _fill_zero(zbuf):
  w = zbuf.shape[-1]
  for i in range(ZR):
    for q in range(w // 16):
      zbuf[i, pl.ds(q * 16, 16)] = jnp.zeros((16,), jnp.float32)


def _zero_rows(zbuf, accum, sid):
  zr = zbuf.shape[0]

  @pl.when(sid < NS - 1)
  def _():
    for k in range(SLAB // zr):
      pltpu.sync_copy(zbuf, accum.at[pl.ds(sid * SLAB + k * zr, zr)])

  @pl.when(sid == NS - 1)
  def _():
    for k in range(SLAB_LAST // zr):
      pltpu.sync_copy(zbuf, accum.at[pl.ds((NS - 1) * SLAB + k * zr, zr)])


def _writeout_rows(accum, out_hbm, sid, roff):
  @pl.when(sid < NS - 1)
  def _():
    pltpu.sync_copy(accum.at[pl.ds(sid * SLAB, SLAB)],
                    out_hbm.at[pl.ds(roff + sid * SLAB, SLAB)])

  @pl.when(sid == NS - 1)
  def _():
    pltpu.sync_copy(accum.at[pl.ds((NS - 1) * SLAB, SLAB_LAST)],
                    out_hbm.at[pl.ds(roff + (NS - 1) * SLAB, SLAB_LAST)])


def _zero_table(tab):
  def body(i, _):
    tab[pl.ds(i * 16, 16)] = jnp.zeros((16,), jnp.float32)
    return 0

  lax.fori_loop(0, N // 16, body, 0)


SAGE_CH = EPW // K


def _make_sage(with_deg):
  out_type = [jax.ShapeDtypeStruct((NC * N, D), jnp.float32)]
  if with_deg:
    out_type.append(jax.ShapeDtypeStruct((NW * N,), jnp.float32))
  scratch = [
      pltpu.VMEM((EPW,), jnp.int32),
      pltpu.VMEM((SAGE_CH, K), jnp.int32),
      pltpu.VMEM((K, D), jnp.float32),
      pltpu.VMEM((K, D), jnp.float32),
      pltpu.VMEM_SHARED((N, D), jnp.float32),
      pltpu.SemaphoreType.DMA,
      pltpu.SemaphoreType.DMA,
      pltpu.SemaphoreType.DMA,
      pltpu.SemaphoreType.DMA,
  ]
  if with_deg:
    scratch.append(pltpu.VMEM((N,), jnp.float32))

  def body(x_hbm, src_hbm, dst2_hbm, p_out, *rest):
    if with_deg:
      (deg_out, src_all, dst2d, rows0, rows1, accum,
       g0, g1, s0, s1, degt) = rest
    else:
      (src_all, dst2d, rows0, rows1, accum, g0, g1, s0, s1) = rest
    cid = lax.axis_index("c")
    sid = lax.axis_index("s")
    wid = cid * NS + sid

    pltpu.sync_copy(src_hbm.at[pl.ds(wid * EPW, EPW)], src_all)
    pltpu.sync_copy(dst2_hbm.at[pl.ds(wid * SAGE_CH, SAGE_CH)], dst2d)
    if with_deg:
      _zero_table(degt)
    ones16 = jnp.ones((16,), jnp.float32)

    def zrow(r, _):
      for q in range(D // 16):
        rows0[r, pl.ds(q * 16, 16)] = jnp.zeros((16,), jnp.float32)
      return 0

    lax.fori_loop(0, K, zrow, 0)
    _zero_rows(rows0, accum, sid)
    plsc.subcore_barrier()

    def gather(ch, rows, gsem):
      pltpu.async_copy(x_hbm.at[src_all.at[pl.ds(ch * K, K)]], rows, gsem)

    def wait_bytes(sem):
      pltpu.make_async_copy(x_hbm.at[pl.ds(0, K)], rows0, sem).wait()

    def scatter(ch, rows, ssem):
      pltpu.async_copy(rows, accum.at[dst2d.at[ch]], ssem, add=True)
      if with_deg:
        for k in range(K // 16):
          d16 = dst2d[ch, pl.ds(k * 16, 16)]
          plsc.addupdate_scatter(degt, [d16], ones16)

    gather(0, rows0, g0)
    gather(1, rows1, g1)

    def pair(j, _):
      wait_bytes(g0)
      scatter(2 * j, rows0, s0)
      wait_bytes(g1)
      scatter(2 * j + 1, rows1, s1)
      wait_bytes(s0)
      gather(2 * j + 2, rows0, g0)

      @pl.when(j < SAGE_CH // 2 - 1)
      def _():
        wait_bytes(s1)
        gather(2 * j + 3, rows1, g1)

      return 0

    lax.fori_loop(0, SAGE_CH // 2, pair, 0)
    wait_bytes(g0)
    scatter(SAGE_CH - 1, rows0, s0)
    wait_bytes(s0)
    wait_bytes(s1)
    plsc.subcore_barrier()

    _writeout_rows(accum, p_out, sid, cid * N)
    if with_deg:
      pltpu.sync_copy(degt, deg_out.at[pl.ds(wid * N, N)])

  return pl.kernel(body, out_type=tuple(out_type), mesh=_mesh,
                   scratch_types=tuple(scratch), compiler_params=_sc_params)


_sage_deg = _make_sage(True)
_sage = _make_sage(False)


KG = 80
GAT_CHK = EPC // KG


def _gat_kernel():
  out_type = (jax.ShapeDtypeStruct((4 * N, HD), jnp.float32),
              jax.ShapeDtypeStruct((NW * N,), jnp.float32))
  scratch = (
      pltpu.VMEM((N,), jnp.float32),
      pltpu.VMEM((N,), jnp.float32),
      pltpu.VMEM((EPC,), jnp.int32),
      pltpu.VMEM((GAT_CHK, KG), jnp.int32),
      pltpu.VMEM((KG, HD), jnp.float32),
      pltpu.VMEM((KG, HD), jnp.float32),
      pltpu.VMEM((KG,), jnp.float32),
      pltpu.VMEM((KG,), jnp.float32),
      pltpu.VMEM((ZR, HD), jnp.float32),
      pltpu.VMEM((N,), jnp.float32),
      pltpu.VMEM_SHARED((N, HD), jnp.float32),
      pltpu.SemaphoreType.DMA,
      pltpu.SemaphoreType.DMA,
      pltpu.SemaphoreType.DMA,
      pltpu.SemaphoreType.DMA,
  )

  def body(h_hbm, src_hbm, dst2_hbm, asd_hbm, o_out, den_out,
           as_t, ad_t, src_all, dst2d, rows0, rows1, eeb0, eeb1,
           zbuf, dent, accum, g0, g1, s0, s1):
    cid = lax.axis_index("c")
    sid = lax.axis_index("s")
    wid = cid * NS + sid

    pltpu.sync_copy(asd_hbm.at[pl.ds(cid * N, N)], as_t)
    pltpu.sync_copy(asd_hbm.at[pl.ds((2 + cid) * N, N)], ad_t)
    pltpu.sync_copy(src_hbm.at[pl.ds(sid * EPC, EPC)], src_all)
    pltpu.sync_copy(dst2_hbm.at[pl.ds(sid * GAT_CHK, GAT_CHK)], dst2d)
    _fill_zero(zbuf)
    _zero_table(dent)

    def mx_body(i, mv):
      return jnp.maximum(mv, as_t[pl.ds(i * 16, 16)])

    mv = lax.fori_loop(0, N // 16, mx_body,
                       jnp.full((16,), -3.4e38, jnp.float32))
    msv = jnp.full((16,), jnp.max(mv, axis=0), jnp.float32)

    def adjust_src(delta):
      dv = jnp.full((16,), delta, jnp.int32)

      @plsc.parallel_loop(0, EPC // 16, unroll=4)
      def _(i):
        src_all[pl.ds(i * 16, 16)] = src_all[pl.ds(i * 16, 16)] + dv

    for f in range(2):
      _zero_rows(zbuf, accum, sid)
      adjust_src(cid * 2 * N if f == 0 else N)
      plsc.subcore_barrier()
      roff = (cid * 2 + f) * N
      off16 = jnp.full((16,), roff, jnp.int32)

      def scal(ch, eeb):
        for k in range(KG // 16):
          s16 = src_all[pl.ds(ch * KG + k * 16, 16)] - off16
          d16 = dst2d[ch, pl.ds(k * 16, 16)]
          es = plsc.load_gather(as_t, [s16])
          ad = plsc.load_gather(ad_t, [d16])
          t = es + ad
          e = jnp.where(t > 0, t, 0.2 * t)
          c0 = msv + ad
          cc = jnp.where(c0 > 0, c0, 0.2 * c0)
          ee = jnp.exp(e - cc)
          if f == 0:
            plsc.addupdate_scatter(dent, [d16], ee)
          eeb[pl.ds(k * 16, 16)] = ee

      def gather(ch, rows, gsem):
        pltpu.async_copy(h_hbm.at[src_all.at[pl.ds(ch * KG, KG)]],
                         rows, gsem)

      def wait_bytes(sem):
        pltpu.make_async_copy(h_hbm.at[pl.ds(0, KG)], rows0, sem).wait()

      def scale_scatter(ch, eeb, rows, ssem):
        @plsc.parallel_loop(0, KG, unroll=8)
        def _(r):
          av = plsc.load_gather(eeb, [jnp.full((16,), r, jnp.int32)])
          for q in range(HD // 16):
            rows[r, pl.ds(q * 16, 16)] = rows[r, pl.ds(q * 16, 16)] * av

        pltpu.async_copy(rows, accum.at[dst2d.at[ch]], ssem, add=True)

      scal(0, eeb0)
      gather(0, rows0, g0)
      scal(1, eeb1)
      gather(1, rows1, g1)

      def pair(j, _):
        wait_bytes(g0)
        scale_scatter(2 * j, eeb0, rows0, s0)
        wait_bytes(g1)
        scale_scatter(2 * j + 1, eeb1, rows1, s1)

        @pl.when(j < GAT_CHK // 2 - 1)
        def _():
          scal(2 * j + 2, eeb0)
          wait_bytes(s0)
          gather(2 * j + 2, rows0, g0)
          scal(2 * j + 3, eeb1)
          wait_bytes(s1)
          gather(2 * j + 3, rows1, g1)

        return 0

      lax.fori_loop(0, GAT_CHK // 2, pair, 0)
      wait_bytes(s0)
      wait_bytes(s1)
      plsc.subcore_barrier()

      _writeout_rows(accum, o_out, sid, roff)
      if f == 0:
        plsc.subcore_barrier()
    pltpu.sync_copy(dent, den_out.at[pl.ds(wid * N, N)])

  return pl.kernel(body, out_type=out_type, mesh=_mesh,
                   scratch_types=scratch, compiler_params=_sc_params)


_gat = _gat_kernel()


R = 1000
GRID = N // R
_f32 = jnp.float32


def _cat(a, b):
  return jnp.concatenate([a, b], axis=-1)


def _tc1_body(pos, p, degp, wl, wr, b, x1):
  cnt = jnp.maximum(jnp.sum(degp[...], axis=1, keepdims=True), 1.0)
  aggr = (p[0] + p[1]) / cnt
  y = (jnp.dot(aggr, wl[...], preferred_element_type=_f32)
       + jnp.dot(pos[...], wr[...], preferred_element_type=_f32) + b[...])
  x1[...] = jnp.maximum(y, 0.0)


def _tc2_body(x1, q, degp, wl, wr, b, wg, ats, atd, h, asd):
  cnt = jnp.maximum(jnp.sum(degp[...], axis=1, keepdims=True), 1.0)
  aggr = (q[0] + q[1]) / cnt
  x1v = x1[...]
  x2 = jnp.maximum(
      jnp.dot(aggr, wl[...], preferred_element_type=_f32)
      + jnp.dot(x1v, wr[...], preferred_element_type=_f32) + b[...], 0.0)
  wgv = wg[...]
  hcat = (jnp.dot(x1v, wgv[:D, :], preferred_element_type=_f32)
          + jnp.dot(x2, wgv[D:, :], preferred_element_type=_f32))
  h0 = hcat[:, :D]
  h1 = hcat[:, D:]
  for i in range(4):
    h[i] = hcat[:, i * HD:(i + 1) * HD]
  atsv = ats[...]
  atdv = atd[...]
  asd[:, 0:1] = jnp.sum(h0 * atsv[0:1, :], axis=-1, keepdims=True)
  asd[:, 1:2] = jnp.sum(h1 * atsv[1:2, :], axis=-1, keepdims=True)
  asd[:, 2:3] = jnp.sum(h0 * atdv[0:1, :], axis=-1, keepdims=True)
  asd[:, 3:4] = jnp.sum(h1 * atdv[1:2, :], axis=-1, keepdims=True)


_BN_SCALE = float(1.0 / (1.0 + EPS) ** 0.5)


def _tc3_body(o, denp, bg, batch, wd1, bd1, g1, be1, wd2, bd2, g2, be2,
              wd3, bd3, g3, be3, z, am, gs, gc):
  i = pl.program_id(0)
  dv = denp[...]
  den0 = jnp.sum(dv[:, :NS], axis=1, keepdims=True)
  den1 = jnp.sum(dv[:, NS:], axis=1, keepdims=True)
  o0 = _cat(o[0], o[1])
  o1 = _cat(o[2], o[3])
  out = 0.5 * (o0 / (den0 + 1e-16) + o1 / (den1 + 1e-16)) + bg[...]
  gid = lax.broadcasted_iota(jnp.int32, (R, B), 1)
  oh = (batch[...] == gid).astype(_f32)
  gs_inc = lax.dot_general(oh, out, (((0,), (0,)), ((), ())),
                           preferred_element_type=_f32)
  gc_inc = lax.dot_general(oh, jnp.ones((R, 1), _f32),
                           (((0,), (0,)), ((), ())),
                           preferred_element_type=_f32)

  @pl.when(i == 0)
  def _():
    gs[...] = jnp.zeros_like(gs)
    gc[...] = jnp.zeros_like(gc)

  gs[...] += gs_inc
  gc[...] += gc_inc

  @pl.when(i == GRID - 1)
  def _():
    zv = gs[...] / jnp.maximum(gc[...], 1.0)

    def bn(x, g, bb):
      return x * _BN_SCALE * g[...] + bb[...]

    y = jnp.maximum(
        bn(jnp.dot(zv, wd1[...], preferred_element_type=_f32) + bd1[...],
           g1, be1), 0.0)
    y = jnp.maximum(
        bn(jnp.dot(y, wd2[...], preferred_element_type=_f32) + bd2[...],
           g2, be2), 0.0)
    y = bn(jnp.dot(y, wd3[...], preferred_element_type=_f32) + bd3[...],
           g3, be3)
    z[...] = y
    mx = jnp.max(y, axis=-1, keepdims=True)
    ii = lax.broadcasted_iota(jnp.int32, (B, OUT), 1)
    am[...] = jnp.min(jnp.where(y >= mx, ii, OUT), axis=-1, keepdims=True)


def _full(shape):
  return pl.BlockSpec(shape, lambda i: tuple(0 for _ in shape))


_tc1 = pl.pallas_call(
    _tc1_body,
    grid=(GRID,),
    in_specs=[
        pl.BlockSpec((R, D), lambda i: (i, 0)),
        pl.BlockSpec((NC, R, D), lambda i: (0, i, 0)),
        pl.BlockSpec((R, NW), lambda i: (i, 0)),
        _full((D, D)), _full((D, D)), _full((1, D)),
    ],
    out_specs=pl.BlockSpec((R, D), lambda i: (i, 0)),
    out_shape=jax.ShapeDtypeStruct((N, D), _f32),
)

_tc2 = pl.pallas_call(
    _tc2_body,
    grid=(GRID,),
    in_specs=[
        pl.BlockSpec((R, D), lambda i: (i, 0)),
        pl.BlockSpec((NC, R, D), lambda i: (0, i, 0)),
        pl.BlockSpec((R, NW), lambda i: (i, 0)),
        _full((D, D)), _full((D, D)), _full((1, D)),
        _full((2 * D, 2 * D)), _full((HEADS, D)), _full((HEADS, D)),
    ],
    out_specs=[
        pl.BlockSpec((4, R, HD), lambda i: (0, i, 0)),
        pl.BlockSpec((R, 4), lambda i: (i, 0)),
    ],
    out_shape=[
        jax.ShapeDtypeStruct((4, N, HD), _f32),
        jax.ShapeDtypeStruct((N, 4), _f32),
    ],
)

_tc3 = pl.pallas_call(
    _tc3_body,
    grid=(GRID,),
    in_specs=[
        pl.BlockSpec((4, R, HD), lambda i: (0, i, 0)),
        pl.BlockSpec((R, NW), lambda i: (i, 0)),
        _full((1, D)),
        pl.BlockSpec((R, 1), lambda i: (i, 0)),
        _full((D, 2 * D)), _full((1, 2 * D)), _full((1, 2 * D)),
        _full((1, 2 * D)),
        _full((2 * D, D)), _full((1, D)), _full((1, D)), _full((1, D)),
        _full((D, OUT)), _full((1, OUT)), _full((1, OUT)), _full((1, OUT)),
    ],
    out_specs=[
        pl.BlockSpec((B, OUT), lambda i: (0, 0)),
        pl.BlockSpec((B, 1), lambda i: (0, 0)),
    ],
    out_shape=[
        jax.ShapeDtypeStruct((B, OUT), _f32),
        jax.ShapeDtypeStruct((B, 1), jnp.int32),
    ],
    scratch_shapes=[
        pltpu.VMEM((B, D), _f32),
        pltpu.VMEM((B, 1), _f32),
    ],
    compiler_params=pltpu.CompilerParams(
        dimension_semantics=("arbitrary",)),
)


def kernel(pos, edge_index, batch, W1_l, W1_r, b1, W2_l, W2_r, b2, W_gat,
           att_src, att_dst, b_gat, Wd1, bd1, g1, be1, Wd2, bd2, g2, be2,
           Wd3, bd3, g3, be3):
  src = edge_index[0]
  dst = edge_index[1]
  dst2s = dst.reshape(E // K, K)
  dst2g = dst.reshape(E // KG, KG)

  p, degp = _sage_deg(pos, src, dst2s)
  p = p.reshape(NC, N, D)
  degpT = jnp.transpose(degp.reshape(NW, N))
  x1 = _tc1(pos, p, degpT, W1_l, W1_r, b1.reshape(1, D))

  q = _sage(x1, src, dst2s)[0].reshape(NC, N, D)
  h, asd = _tc2(x1, q, degpT, W2_l, W2_r, b2.reshape(1, D),
                W_gat, att_src, att_dst)

  asdT = jnp.transpose(asd).reshape(4 * N)
  o, denp = _gat(h.reshape(4 * N, HD), src, dst2g, asdT)
  o = o.reshape(4, N, HD)
  denpT = jnp.transpose(denp.reshape(NW, N))

  z, am = _tc3(o, denpT, b_gat.reshape(1, D), batch.reshape(N, 1),
               Wd1, bd1.reshape(1, -1), g1.reshape(1, -1), be1.reshape(1, -1),
               Wd2, bd2.reshape(1, -1), g2.reshape(1, -1), be2.reshape(1, -1),
               Wd3, bd3.reshape(1, -1), g3.reshape(1, -1), be3.reshape(1, -1))
  return (z, am.reshape(B))
